# Initial kernel scaffold; baseline (speedup 1.0000x reference)
import functools

import jax
import jax.numpy as jnp
from jax import lax
from jax.experimental import pallas as pl
from jax.experimental.pallas import tpu as pltpu
from jax.experimental.pallas import tpu_sc as plsc

N = 50000
E = 800000
NGRAPH = 64

NP = 50048
RB = 128
NBLK_TC = NP // RB
ETOT = E + N
EPAD = 851968
EPADV = EPAD - ETOT
EB = 128
NTILE = 16
NCORE = 2
R16 = NP // NTILE

_f32 = jnp.float32
_i32 = jnp.int32


def _sc_body_factory(hc, mode_a):
    def body(src_ref, dst_ref, htab_ref, atab_ref, zh_ref, ze_ref,
             acch_ref, acce_ref,
             sidx, didx, sidxt, didxt, hrows, ars, ard, exrows,
             acchs, acces, sem):
        c = lax.axis_index("c")
        s = lax.axis_index("s")
        r0 = s * R16
        iota16 = lax.iota(_i32, 16)
        zeros16 = jnp.zeros((16,), _f32)

        for g in range(EB // 16):
            eids = iota16 + g * 16
            for colz in range(hc, 8):
                plsc.store_scatter(exrows, [eids, jnp.full((16,), colz, _i32)],
                                   zeros16)

        pltpu.sync_copy(zh_ref.at[pl.ds(r0, R16)], acchs.at[pl.ds(r0, R16)])
        pltpu.sync_copy(ze_ref.at[pl.ds(r0, R16)], acces.at[pl.ds(r0, R16)])
        plsc.subcore_barrier()

        if mode_a:
            nblk = EPAD // NTILE // EB
            e_base = s * (EPAD // NTILE)
            tab_off = c * NP
        else:
            nblk = EPAD // (NTILE * NCORE) // EB
            wid = c * NTILE + s
            e_base = wid * (EPAD // (NTILE * NCORE))
            tab_off = None

        @pl.loop(0, nblk)
        def _edge_block(it):
            base = e_base + it * EB
            pltpu.sync_copy(src_ref.at[pl.ds(base, EB)], sidx)
            pltpu.sync_copy(dst_ref.at[pl.ds(base, EB)], didx)
            if mode_a:
                for g in range(EB // 16):
                    sl = pl.ds(g * 16, 16)
                    sidxt[sl] = sidx[sl] + tab_off
                    didxt[sl] = didx[sl] + tab_off
                gsrc, gdst = sidxt, didxt
            else:
                gsrc, gdst = sidx, didx
            pltpu.async_copy(htab_ref.at[gsrc], hrows, sem).wait()
            pltpu.async_copy(atab_ref.at[gsrc], ars, sem).wait()
            pltpu.async_copy(atab_ref.at[gdst], ard, sem).wait()
            for g in range(EB // 16):
                eids = iota16 + g * 16
                exs = []
                for hh in range(hc):
                    col_s = jnp.full((16,), hh, _i32)
                    col_d = jnp.full((16,), 4 + hh, _i32)
                    asv = plsc.load_gather(ars, [eids, col_s])
                    adv = plsc.load_gather(ard, [eids, col_d])
                    al = asv + adv
                    al = jnp.where(al > 0.0, al, al * 0.2)
                    exv = jnp.exp(al)
                    plsc.store_scatter(exrows, [eids, col_s], exv)
                    exs.append(exv)
                for col in range(32):
                    colv = jnp.full((16,), col, _i32)
                    hv = plsc.load_gather(hrows, [eids, colv])
                    hv = hv * exs[col * hc // 32]
                    plsc.store_scatter(hrows, [eids, colv], hv)
            pltpu.sync_copy(hrows, acchs.at[didx], add=True)
            pltpu.sync_copy(exrows, acces.at[didx], add=True)

        plsc.subcore_barrier()
        pltpu.sync_copy(acchs.at[pl.ds(r0, R16)],
                        acch_ref.at[c, pl.ds(r0, R16)])
        pltpu.sync_copy(acces.at[pl.ds(r0, R16)],
                        acce_ref.at[c, pl.ds(r0, R16)])

    return body


def _make_sc_layer(hc, mode_a):
    mesh = plsc.VectorSubcoreMesh(core_axis_name="c", subcore_axis_name="s")
    return pl.kernel(
        _sc_body_factory(hc, mode_a),
        out_type=(jax.ShapeDtypeStruct((NCORE, NP, 32), _f32),
                  jax.ShapeDtypeStruct((NCORE, NP, 8), _f32)),
        mesh=mesh,
        scratch_types=[
            pltpu.VMEM((EB,), _i32),
            pltpu.VMEM((EB,), _i32),
            pltpu.VMEM((EB,), _i32),
            pltpu.VMEM((EB,), _i32),
            pltpu.VMEM((EB, 32), _f32),
            pltpu.VMEM((EB, 8), _f32),
            pltpu.VMEM((EB, 8), _f32),
            pltpu.VMEM((EB, 8), _f32),
            pltpu.VMEM_SHARED((NP, 32), _f32),
            pltpu.VMEM_SHARED((NP, 8), _f32),
            pltpu.SemaphoreType.DMA,
        ],
        name=f"gat_edge_hc{hc}_{'A' if mode_a else 'B'}",
    )


def _full(shape):
    return pl.BlockSpec(shape, lambda i: tuple(0 for _ in shape))


def _rows(width):
    return pl.BlockSpec((RB, width), lambda i: (i, 0))


def _chunk_tabs(width):
    return pl.BlockSpec((2, RB, width), lambda i: (0, i, 0))


def _tables_4h(h, asr, adt):
    htab = jnp.stack([h[:, :32], h[:, 32:]])
    z2 = jnp.zeros((RB, 2), _f32)
    a0 = jnp.concatenate([asr[:, 0:2], z2, adt[:, 0:2], z2], axis=1)
    a1 = jnp.concatenate([asr[:, 2:4], z2, adt[:, 2:4], z2], axis=1)
    return htab, jnp.stack([a0, a1])


def _k1_body(xf_ref, xb_ref, wf_ref, asf_ref, adf_ref,
             wb_ref, asb_ref, adb_ref,
             hf_ref, af_ref, hb_ref, ab_ref):
    for x_ref, w_ref, as_ref, ad_ref, h_out, a_out in (
            (xf_ref, wf_ref, asf_ref, adf_ref, hf_ref, af_ref),
            (xb_ref, wb_ref, asb_ref, adb_ref, hb_ref, ab_ref)):
        x = x_ref[...]
        h = jnp.dot(x, w_ref[...], preferred_element_type=_f32)
        asr = jnp.dot(h, as_ref[...], preferred_element_type=_f32)
        adt = jnp.dot(h, ad_ref[...], preferred_element_type=_f32)
        htab, atab = _tables_4h(h, asr, adt)
        h_out[...] = htab
        a_out[...] = atab


def _combine_4h(acch, acce, bias):
    cols = []
    for c in range(2):
        for hh in range(2):
            num = acch[c][:, hh * 16:(hh + 1) * 16]
            den = acce[c][:, hh:hh + 1] + 1e-16
            cols.append(num / den)
    v = jnp.concatenate(cols, axis=1) + bias
    return jnp.where(v > 0.0, v, jnp.exp(v) - 1.0)


def _k2_body(ahf_ref, aef_ref, ahb_ref, aeb_ref,
             bf_ref, wf_ref, asf_ref, adf_ref,
             bb_ref, wb_ref, asb_ref, adb_ref,
             hf_ref, af_ref, hb_ref, ab_ref):
    for ah_ref, ae_ref, b_ref, w_ref, as_ref, ad_ref, h_out, a_out in (
            (ahf_ref, aef_ref, bf_ref, wf_ref, asf_ref, adf_ref, hf_ref, af_ref),
            (ahb_ref, aeb_ref, bb_ref, wb_ref, asb_ref, adb_ref, hb_ref, ab_ref)):
        xin = _combine_4h(ah_ref[...], ae_ref[...], b_ref[...])
        h = jnp.dot(xin, w_ref[...], preferred_element_type=_f32)
        asr = jnp.dot(h, as_ref[...], preferred_element_type=_f32)
        adt = jnp.dot(h, ad_ref[...], preferred_element_type=_f32)
        htab, atab = _tables_4h(h, asr, adt)
        h_out[...] = htab
        a_out[...] = atab


def _k3_body(ahf_ref, aef_ref, ahb_ref, aeb_ref,
             bf_ref, wf_ref, asf_ref, adf_ref,
             bb_ref, wb_ref, asb_ref, adb_ref,
             hf_ref, af_ref, hb_ref, ab_ref):
    z3 = jnp.zeros((RB, 3), _f32)
    for ah_ref, ae_ref, b_ref, w_ref, as_ref, ad_ref, h_out, a_out in (
            (ahf_ref, aef_ref, bf_ref, wf_ref, asf_ref, adf_ref, hf_ref, af_ref),
            (ahb_ref, aeb_ref, bb_ref, wb_ref, asb_ref, adb_ref, hb_ref, ab_ref)):
        xin = _combine_4h(ah_ref[...], ae_ref[...], b_ref[...])
        h = jnp.dot(xin, w_ref[...], preferred_element_type=_f32)
        asr = jnp.dot(h, as_ref[...], preferred_element_type=_f32)
        adt = jnp.dot(h, ad_ref[...], preferred_element_type=_f32)
        h_out[...] = h
        a_out[...] = jnp.concatenate([asr, z3, adt, z3], axis=1)


def _kpool_body(ahf_ref, aef_ref, ahb_ref, aeb_ref,
                bf_ref, bb_ref, batch_ref,
                hnode_ref, gpool_ref, accp, accc):
    i = pl.program_id(0)
    parts = []
    for ah_ref, ae_ref, b_ref in ((ahf_ref, aef_ref, bf_ref),
                                  (ahb_ref, aeb_ref, bb_ref)):
        ah = ah_ref[...]
        ae = ae_ref[...]
        num = ah[0] + ah[1]
        den = ae[0][:, 0:1] + ae[1][:, 0:1] + 1e-16
        parts.append(num / den + b_ref[...])
    hn = jnp.concatenate(parts, axis=1)
    hnode_ref[...] = hn

    row = lax.broadcasted_iota(_i32, (RB, 1), 0) + i * RB
    valid = row < N
    bt = batch_ref[0, 0, :].reshape(RB, 1)
    gid = lax.broadcasted_iota(_i32, (RB, NGRAPH), 1)
    oh = jnp.where((bt == gid) & valid, 1.0, 0.0).astype(_f32)
    contrib = lax.dot_general(oh, hn, (((0,), (0,)), ((), ())),
                              preferred_element_type=_f32)
    cnt = jnp.sum(oh, axis=0).reshape(NGRAPH, 1)
    newp = jnp.where(i == 0, contrib, accp[...] + contrib)
    newc = jnp.where(i == 0, cnt, accc[...] + cnt)
    accp[...] = newp
    accc[...] = newc

    @pl.when(i == NBLK_TC - 1)
    def _():
        gpool_ref[...] = newp / jnp.clip(newc, 1.0)


def _run(xf, xb, src, dst, batchp, pp):
    zh = jnp.zeros((NP, 32), _f32)
    ze = jnp.zeros((NP, 8), _f32)

    k1 = pl.pallas_call(
        _k1_body,
        grid=(NBLK_TC,),
        in_specs=[_rows(8), _rows(8),
                  _full((8, 64)), _full((64, 4)), _full((64, 4)),
                  _full((8, 64)), _full((64, 4)), _full((64, 4))],
        out_specs=[_chunk_tabs(32), _chunk_tabs(8),
                   _chunk_tabs(32), _chunk_tabs(8)],
        out_shape=[jax.ShapeDtypeStruct((2, NP, 32), _f32),
                   jax.ShapeDtypeStruct((2, NP, 8), _f32),
                   jax.ShapeDtypeStruct((2, NP, 32), _f32),
                   jax.ShapeDtypeStruct((2, NP, 8), _f32)],
    )
    hf1, af1, hb1, ab1 = k1(xf, xb, pp['f1W'], pp['f1As'], pp['f1Ad'],
                            pp['b1W'], pp['b1As'], pp['b1Ad'])

    sc4 = _make_sc_layer(2, True)
    sc1 = _make_sc_layer(1, False)

    ahf1, aef1 = sc4(src, dst, hf1.reshape(2 * NP, 32),
                     af1.reshape(2 * NP, 8), zh, ze)
    ahb1, aeb1 = sc4(dst, src, hb1.reshape(2 * NP, 32),
                     ab1.reshape(2 * NP, 8), zh, ze)

    k2 = pl.pallas_call(
        _k2_body,
        grid=(NBLK_TC,),
        in_specs=[_chunk_tabs(32), _chunk_tabs(8),
                  _chunk_tabs(32), _chunk_tabs(8),
                  _full((1, 64)), _full((64, 64)), _full((64, 4)), _full((64, 4)),
                  _full((1, 64)), _full((64, 64)), _full((64, 4)), _full((64, 4))],
        out_specs=[_chunk_tabs(32), _chunk_tabs(8),
                   _chunk_tabs(32), _chunk_tabs(8)],
        out_shape=[jax.ShapeDtypeStruct((2, NP, 32), _f32),
                   jax.ShapeDtypeStruct((2, NP, 8), _f32),
                   jax.ShapeDtypeStruct((2, NP, 32), _f32),
                   jax.ShapeDtypeStruct((2, NP, 8), _f32)],
    )
    hf2, af2, hb2, ab2 = k2(ahf1, aef1, ahb1, aeb1,
                            pp['f1b'], pp['f2W'], pp['f2As'], pp['f2Ad'],
                            pp['b1b'], pp['b2W'], pp['b2As'], pp['b2Ad'])

    ahf2, aef2 = sc4(src, dst, hf2.reshape(2 * NP, 32),
                     af2.reshape(2 * NP, 8), zh, ze)
    ahb2, aeb2 = sc4(dst, src, hb2.reshape(2 * NP, 32),
                     ab2.reshape(2 * NP, 8), zh, ze)

    k3 = pl.pallas_call(
        _k3_body,
        grid=(NBLK_TC,),
        in_specs=[_chunk_tabs(32), _chunk_tabs(8),
                  _chunk_tabs(32), _chunk_tabs(8),
                  _full((1, 64)), _full((64, 32)), _full((32, 1)), _full((32, 1)),
                  _full((1, 64)), _full((64, 32)), _full((32, 1)), _full((32, 1))],
        out_specs=[_rows(32), _rows(8), _rows(32), _rows(8)],
        out_shape=[jax.ShapeDtypeStruct((NP, 32), _f32),
                   jax.ShapeDtypeStruct((NP, 8), _f32),
                   jax.ShapeDtypeStruct((NP, 32), _f32),
                   jax.ShapeDtypeStruct((NP, 8), _f32)],
    )
    hf3, af3, hb3, ab3 = k3(ahf2, aef2, ahb2, aeb2,
                            pp['f2b'], pp['f3W'], pp['f3As'], pp['f3Ad'],
                            pp['b2b'], pp['b3W'], pp['b3As'], pp['b3Ad'])

    ahf3, aef3 = sc1(src, dst, hf3, af3, zh, ze)
    ahb3, aeb3 = sc1(dst, src, hb3, ab3, zh, ze)

    kpool = pl.pallas_call(
        _kpool_body,
        grid=(NBLK_TC,),
        in_specs=[_chunk_tabs(32), _chunk_tabs(8),
                  _chunk_tabs(32), _chunk_tabs(8),
                  _full((1, 32)), _full((1, 32)),
                  pl.BlockSpec((1, 1, RB), lambda i: (i, 0, 0))],
        out_specs=[_rows(64), pl.BlockSpec((NGRAPH, NGRAPH), lambda i: (0, 0))],
        out_shape=[jax.ShapeDtypeStruct((N, 64), _f32),
                   jax.ShapeDtypeStruct((NGRAPH, NGRAPH), _f32)],
        scratch_shapes=[pltpu.VMEM((NGRAPH, NGRAPH), _f32),
                        pltpu.VMEM((NGRAPH, 1), _f32)],
    )
    h_node, g_pool = kpool(ahf3, aef3, ahb3, aeb3,
                           pp['f3b'], pp['b3b'], batchp)
    return h_node, g_pool


def _attn_mat(a, heads, out_ch):
    if heads == 1:
        return a.reshape(out_ch, 1)
    eye = jnp.repeat(jnp.eye(heads, dtype=_f32), out_ch, axis=0)
    return a.reshape(heads * out_ch, 1) * eye


def kernel(x, params, edge_index, batch):
    ei = edge_index.astype(_i32)
    loops = jnp.arange(N, dtype=_i32)
    padv = jnp.full((EPADV,), N, _i32)
    src = jnp.concatenate([ei[0], loops, padv])
    dst = jnp.concatenate([ei[1], loops, padv])

    xf = jnp.pad(x[:, jnp.array([0, 1, 3])], ((0, NP - N), (0, 5)))
    xb = jnp.pad(x[:, jnp.array([0, 2, 4])], ((0, NP - N), (0, 5)))
    batchp = jnp.pad(batch.astype(_i32), (0, NP - N)).reshape(NBLK_TC, 1, RB)

    pp = {}
    for name, heads, out_ch in (('f1', 4, 16), ('f2', 4, 16), ('f3', 1, 32),
                                ('b1', 4, 16), ('b2', 4, 16), ('b3', 1, 32)):
        p = params[name]
        w = p['W']
        if w.shape[0] == 3:
            w = jnp.pad(w, ((0, 5), (0, 0)))
        pp[name + 'W'] = w
        pp[name + 'As'] = _attn_mat(p['a_s'], heads, out_ch)
        pp[name + 'Ad'] = _attn_mat(p['a_d'], heads, out_ch)
        pp[name + 'b'] = p['b'].reshape(1, -1)

    return _run(xf, xb, src, dst, batchp, pp)


# speedup vs baseline: 18.5586x; 18.5586x over previous
import jax
import jax.numpy as jnp
from jax import lax
from jax.experimental import pallas as pl
from jax.experimental.pallas import tpu as pltpu
from jax.experimental.pallas import tpu_sc as plsc

N = 50000
E = 800000
NGRAPH = 64

NP = 50048
RB = 128
NBLK_TC = NP // RB
ETOT = E + N
EPAD = 851968
EPADV = EPAD - ETOT
EB = 128
NTILE = 16
NCORE = 2
R16 = NP // NTILE

_f32 = jnp.float32
_i32 = jnp.int32


def _sc_body_factory(hc, mode_a):
    def body(src_ref, dst_ref, htab_ref, atab_ref, zh_ref,
             acch_ref, acce_ref,
             sidx, didx, sidxt, didxt, hrows, ars, ard,
             acchs, sem):
        c = lax.axis_index("c")
        s = lax.axis_index("s")
        r0 = s * R16
        iota16 = lax.iota(_i32, 16)
        zeros16 = jnp.zeros((16,), _f32)

        if mode_a:
            nblk = EPAD // NTILE // EB
            e_base = s * (EPAD // NTILE)
            tab_off = c * NP
        else:
            nblk = EPAD // (NTILE * NCORE) // EB
            wid = c * NTILE + s
            e_base = wid * (EPAD // (NTILE * NCORE))
            tab_off = None

        def load_indices(base):
            pltpu.sync_copy(src_ref.at[pl.ds(base, EB)], sidx)
            pltpu.sync_copy(dst_ref.at[pl.ds(base, EB)], didx)
            if mode_a:
                for g in range(EB // 16):
                    sl = pl.ds(g * 16, 16)
                    sidxt[sl] = sidx[sl] + tab_off
                    didxt[sl] = didx[sl] + tab_off
                return sidxt, didxt
            return sidx, didx

        def edge_ex(g):
            eids = iota16 + g * 16
            exs = []
            for hh in range(hc):
                asv = plsc.load_gather(ars, [eids, jnp.full((16,), hh, _i32)])
                adv = plsc.load_gather(ard, [eids, jnp.full((16,), 8 + hh, _i32)])
                al = asv + adv
                al = jnp.where(al > 0.0, al, al * 0.2)
                exs.append(jnp.exp(al))
            return eids, exs

        @pl.loop(0, EB)
        def _zrow(r):
            hrows[r, pl.ds(0, 16)] = zeros16
            hrows[r, pl.ds(16, 16)] = zeros16

        pltpu.sync_copy(zh_ref.at[pl.ds(r0, R16)], acchs.at[pl.ds(r0, R16)])
        plsc.subcore_barrier()

        @pl.loop(0, nblk)
        def _denom_block(it):
            gsrc, gdst = load_indices(e_base + it * EB)
            pltpu.async_copy(atab_ref.at[gsrc], ars, sem).wait()
            pltpu.async_copy(atab_ref.at[gdst], ard, sem).wait()
            for g in range(EB // 16):
                eids, exs = edge_ex(g)
                for hh in range(hc):
                    plsc.store_scatter(hrows, [eids, jnp.full((16,), hh, _i32)],
                                       exs[hh])
            pltpu.sync_copy(hrows, acchs.at[didx], add=True)

        plsc.subcore_barrier()
        pltpu.sync_copy(acchs.at[pl.ds(r0, R16)],
                        acce_ref.at[c, pl.ds(r0, R16)])
        plsc.subcore_barrier()

        pltpu.sync_copy(zh_ref.at[pl.ds(r0, R16)], acchs.at[pl.ds(r0, R16)])
        plsc.subcore_barrier()

        @pl.loop(0, nblk)
        def _feat_block(it):
            gsrc, gdst = load_indices(e_base + it * EB)
            pltpu.async_copy(htab_ref.at[gsrc], hrows, sem).wait()
            pltpu.async_copy(atab_ref.at[gsrc], ars, sem).wait()
            pltpu.async_copy(atab_ref.at[gdst], ard, sem).wait()
            for g in range(EB // 16):
                eids, exs = edge_ex(g)
                for col in range(32):
                    colv = jnp.full((16,), col, _i32)
                    hv = plsc.load_gather(hrows, [eids, colv])
                    plsc.store_scatter(hrows, [eids, colv],
                                       hv * exs[col * hc // 32])
            pltpu.sync_copy(hrows, acchs.at[didx], add=True)

        plsc.subcore_barrier()
        pltpu.sync_copy(acchs.at[pl.ds(r0, R16)],
                        acch_ref.at[c, pl.ds(r0, R16)])

    return body


def _make_sc_layer(hc, mode_a):
    mesh = plsc.VectorSubcoreMesh(core_axis_name="c", subcore_axis_name="s")
    return pl.kernel(
        _sc_body_factory(hc, mode_a),
        out_type=(jax.ShapeDtypeStruct((NCORE, NP, 32), _f32),
                  jax.ShapeDtypeStruct((NCORE, NP, 32), _f32)),
        mesh=mesh,
        scratch_types=[
            pltpu.VMEM((EB,), _i32),
            pltpu.VMEM((EB,), _i32),
            pltpu.VMEM((EB,), _i32),
            pltpu.VMEM((EB,), _i32),
            pltpu.VMEM((EB, 32), _f32),
            pltpu.VMEM((EB, 16), _f32),
            pltpu.VMEM((EB, 16), _f32),
            pltpu.VMEM_SHARED((NP, 32), _f32),
            pltpu.SemaphoreType.DMA,
        ],
        compiler_params=pltpu.CompilerParams(needs_layout_passes=False,
                                             use_tc_tiling_on_sc=False),
        name=f"gat_edge_hc{hc}_{'A' if mode_a else 'B'}",
    )


def _full(shape):
    return pl.BlockSpec(shape, lambda i: tuple(0 for _ in shape))


def _rows(width):
    return pl.BlockSpec((RB, width), lambda i: (i, 0))


def _chunk_tabs(width):
    return pl.BlockSpec((2, RB, width), lambda i: (0, i, 0))


def _tables_4h(h, asr, adt):
    htab = jnp.stack([h[:, :32], h[:, 32:]])
    z6 = jnp.zeros((RB, 6), _f32)
    a0 = jnp.concatenate([asr[:, 0:2], z6, adt[:, 0:2], z6], axis=1)
    a1 = jnp.concatenate([asr[:, 2:4], z6, adt[:, 2:4], z6], axis=1)
    return htab, jnp.stack([a0, a1])


def _k1_body(xf_ref, xb_ref, wf_ref, asf_ref, adf_ref,
             wb_ref, asb_ref, adb_ref,
             hf_ref, af_ref, hb_ref, ab_ref):
    for x_ref, w_ref, as_ref, ad_ref, h_out, a_out in (
            (xf_ref, wf_ref, asf_ref, adf_ref, hf_ref, af_ref),
            (xb_ref, wb_ref, asb_ref, adb_ref, hb_ref, ab_ref)):
        x = x_ref[...]
        h = jnp.dot(x, w_ref[...], preferred_element_type=_f32)
        asr = jnp.dot(h, as_ref[...], preferred_element_type=_f32)
        adt = jnp.dot(h, ad_ref[...], preferred_element_type=_f32)
        htab, atab = _tables_4h(h, asr, adt)
        h_out[...] = htab
        a_out[...] = atab


def _combine_4h(acch, acce, bias):
    cols = []
    for c in range(2):
        for hh in range(2):
            num = acch[c][:, hh * 16:(hh + 1) * 16]
            den = acce[c][:, hh:hh + 1] + 1e-16
            cols.append(num / den)
    v = jnp.concatenate(cols, axis=1) + bias
    return jnp.where(v > 0.0, v, jnp.exp(v) - 1.0)


def _k2_body(ahf_ref, aef_ref, ahb_ref, aeb_ref,
             bf_ref, wf_ref, asf_ref, adf_ref,
             bb_ref, wb_ref, asb_ref, adb_ref,
             hf_ref, af_ref, hb_ref, ab_ref):
    for ah_ref, ae_ref, b_ref, w_ref, as_ref, ad_ref, h_out, a_out in (
            (ahf_ref, aef_ref, bf_ref, wf_ref, asf_ref, adf_ref, hf_ref, af_ref),
            (ahb_ref, aeb_ref, bb_ref, wb_ref, asb_ref, adb_ref, hb_ref, ab_ref)):
        xin = _combine_4h(ah_ref[...], ae_ref[...], b_ref[...])
        h = jnp.dot(xin, w_ref[...], preferred_element_type=_f32)
        asr = jnp.dot(h, as_ref[...], preferred_element_type=_f32)
        adt = jnp.dot(h, ad_ref[...], preferred_element_type=_f32)
        htab, atab = _tables_4h(h, asr, adt)
        h_out[...] = htab
        a_out[...] = atab


def _k3_body(ahf_ref, aef_ref, ahb_ref, aeb_ref,
             bf_ref, wf_ref, asf_ref, adf_ref,
             bb_ref, wb_ref, asb_ref, adb_ref,
             hf_ref, af_ref, hb_ref, ab_ref):
    z7 = jnp.zeros((RB, 7), _f32)
    for ah_ref, ae_ref, b_ref, w_ref, as_ref, ad_ref, h_out, a_out in (
            (ahf_ref, aef_ref, bf_ref, wf_ref, asf_ref, adf_ref, hf_ref, af_ref),
            (ahb_ref, aeb_ref, bb_ref, wb_ref, asb_ref, adb_ref, hb_ref, ab_ref)):
        xin = _combine_4h(ah_ref[...], ae_ref[...], b_ref[...])
        h = jnp.dot(xin, w_ref[...], preferred_element_type=_f32)
        asr = jnp.dot(h, as_ref[...], preferred_element_type=_f32)
        adt = jnp.dot(h, ad_ref[...], preferred_element_type=_f32)
        h_out[...] = h
        a_out[...] = jnp.concatenate([asr, z7, adt, z7], axis=1)


def _kpool_body(ahf_ref, aef_ref, ahb_ref, aeb_ref,
                bf_ref, bb_ref, batch_ref,
                hnode_ref, gpool_ref, accp, accc):
    i = pl.program_id(0)
    parts = []
    for ah_ref, ae_ref, b_ref in ((ahf_ref, aef_ref, bf_ref),
                                  (ahb_ref, aeb_ref, bb_ref)):
        ah = ah_ref[...]
        ae = ae_ref[...]
        num = ah[0] + ah[1]
        den = ae[0][:, 0:1] + ae[1][:, 0:1] + 1e-16
        parts.append(num / den + b_ref[...])
    hn = jnp.concatenate(parts, axis=1)
    hnode_ref[...] = hn

    row = lax.broadcasted_iota(_i32, (RB, 1), 0) + i * RB
    valid = row < N
    bt = batch_ref[0, 0, :].reshape(RB, 1)
    gid = lax.broadcasted_iota(_i32, (RB, NGRAPH), 1)
    oh = jnp.where((bt == gid) & valid, 1.0, 0.0).astype(_f32)
    contrib = lax.dot_general(oh, hn, (((0,), (0,)), ((), ())),
                              preferred_element_type=_f32)
    cnt = jnp.sum(oh, axis=0).reshape(NGRAPH, 1)
    newp = jnp.where(i == 0, contrib, accp[...] + contrib)
    newc = jnp.where(i == 0, cnt, accc[...] + cnt)
    accp[...] = newp
    accc[...] = newc

    @pl.when(i == NBLK_TC - 1)
    def _():
        gpool_ref[...] = newp / jnp.clip(newc, 1.0)


def _run(xf, xb, src, dst, batchp, pp):
    zh = jnp.zeros((NP, 32), _f32)

    k1 = pl.pallas_call(
        _k1_body,
        grid=(NBLK_TC,),
        in_specs=[_rows(8), _rows(8),
                  _full((8, 64)), _full((64, 4)), _full((64, 4)),
                  _full((8, 64)), _full((64, 4)), _full((64, 4))],
        out_specs=[_chunk_tabs(32), _chunk_tabs(16),
                   _chunk_tabs(32), _chunk_tabs(16)],
        out_shape=[jax.ShapeDtypeStruct((2, NP, 32), _f32),
                   jax.ShapeDtypeStruct((2, NP, 16), _f32),
                   jax.ShapeDtypeStruct((2, NP, 32), _f32),
                   jax.ShapeDtypeStruct((2, NP, 16), _f32)],
    )
    hf1, af1, hb1, ab1 = k1(xf, xb, pp['f1W'], pp['f1As'], pp['f1Ad'],
                            pp['b1W'], pp['b1As'], pp['b1Ad'])

    sc4 = _make_sc_layer(2, True)
    sc1 = _make_sc_layer(1, False)

    ahf1, aef1 = sc4(src, dst, hf1.reshape(2 * NP, 32),
                     af1.reshape(2 * NP, 16), zh)
    ahb1, aeb1 = sc4(dst, src, hb1.reshape(2 * NP, 32),
                     ab1.reshape(2 * NP, 16), zh)

    k2 = pl.pallas_call(
        _k2_body,
        grid=(NBLK_TC,),
        in_specs=[_chunk_tabs(32), _chunk_tabs(32),
                  _chunk_tabs(32), _chunk_tabs(32),
                  _full((1, 64)), _full((64, 64)), _full((64, 4)), _full((64, 4)),
                  _full((1, 64)), _full((64, 64)), _full((64, 4)), _full((64, 4))],
        out_specs=[_chunk_tabs(32), _chunk_tabs(16),
                   _chunk_tabs(32), _chunk_tabs(16)],
        out_shape=[jax.ShapeDtypeStruct((2, NP, 32), _f32),
                   jax.ShapeDtypeStruct((2, NP, 16), _f32),
                   jax.ShapeDtypeStruct((2, NP, 32), _f32),
                   jax.ShapeDtypeStruct((2, NP, 16), _f32)],
    )
    hf2, af2, hb2, ab2 = k2(ahf1, aef1, ahb1, aeb1,
                            pp['f1b'], pp['f2W'], pp['f2As'], pp['f2Ad'],
                            pp['b1b'], pp['b2W'], pp['b2As'], pp['b2Ad'])

    ahf2, aef2 = sc4(src, dst, hf2.reshape(2 * NP, 32),
                     af2.reshape(2 * NP, 16), zh)
    ahb2, aeb2 = sc4(dst, src, hb2.reshape(2 * NP, 32),
                     ab2.reshape(2 * NP, 16), zh)

    k3 = pl.pallas_call(
        _k3_body,
        grid=(NBLK_TC,),
        in_specs=[_chunk_tabs(32), _chunk_tabs(32),
                  _chunk_tabs(32), _chunk_tabs(32),
                  _full((1, 64)), _full((64, 32)), _full((32, 1)), _full((32, 1)),
                  _full((1, 64)), _full((64, 32)), _full((32, 1)), _full((32, 1))],
        out_specs=[_rows(32), _rows(16), _rows(32), _rows(16)],
        out_shape=[jax.ShapeDtypeStruct((NP, 32), _f32),
                   jax.ShapeDtypeStruct((NP, 16), _f32),
                   jax.ShapeDtypeStruct((NP, 32), _f32),
                   jax.ShapeDtypeStruct((NP, 16), _f32)],
    )
    hf3, af3, hb3, ab3 = k3(ahf2, aef2, ahb2, aeb2,
                            pp['f2b'], pp['f3W'], pp['f3As'], pp['f3Ad'],
                            pp['b2b'], pp['b3W'], pp['b3As'], pp['b3Ad'])

    ahf3, aef3 = sc1(src, dst, hf3, af3, zh)
    ahb3, aeb3 = sc1(dst, src, hb3, ab3, zh)

    kpool = pl.pallas_call(
        _kpool_body,
        grid=(NBLK_TC,),
        in_specs=[_chunk_tabs(32), _chunk_tabs(32),
                  _chunk_tabs(32), _chunk_tabs(32),
                  _full((1, 32)), _full((1, 32)),
                  pl.BlockSpec((1, 1, RB), lambda i: (i, 0, 0))],
        out_specs=[_rows(64), pl.BlockSpec((NGRAPH, NGRAPH), lambda i: (0, 0))],
        out_shape=[jax.ShapeDtypeStruct((N, 64), _f32),
                   jax.ShapeDtypeStruct((NGRAPH, NGRAPH), _f32)],
        scratch_shapes=[pltpu.VMEM((NGRAPH, NGRAPH), _f32),
                        pltpu.VMEM((NGRAPH, 1), _f32)],
    )
    h_node, g_pool = kpool(ahf3, aef3, ahb3, aeb3,
                           pp['f3b'], pp['b3b'], batchp)
    return h_node, g_pool


def _attn_mat(a, heads, out_ch):
    if heads == 1:
        return a.reshape(out_ch, 1)
    eye = jnp.repeat(jnp.eye(heads, dtype=_f32), out_ch, axis=0)
    return a.reshape(heads * out_ch, 1) * eye


def kernel(x, params, edge_index, batch):
    ei = edge_index.astype(_i32)
    loops = jnp.arange(N, dtype=_i32)
    padv = jnp.full((EPADV,), N, _i32)
    src = jnp.concatenate([ei[0], loops, padv])
    dst = jnp.concatenate([ei[1], loops, padv])

    xf = jnp.pad(x[:, jnp.array([0, 1, 3])], ((0, NP - N), (0, 5)))
    xb = jnp.pad(x[:, jnp.array([0, 2, 4])], ((0, NP - N), (0, 5)))
    batchp = jnp.pad(batch.astype(_i32), (0, NP - N)).reshape(NBLK_TC, 1, RB)

    pp = {}
    for name, heads, out_ch in (('f1', 4, 16), ('f2', 4, 16), ('f3', 1, 32),
                                ('b1', 4, 16), ('b2', 4, 16), ('b3', 1, 32)):
        p = params[name]
        w = p['W']
        if w.shape[0] == 3:
            w = jnp.pad(w, ((0, 5), (0, 0)))
        pp[name + 'W'] = w
        pp[name + 'As'] = _attn_mat(p['a_s'], heads, out_ch)
        pp[name + 'Ad'] = _attn_mat(p['a_d'], heads, out_ch)
        pp[name + 'b'] = p['b'].reshape(1, -1)

    return _run(xf, xb, src, dst, batchp, pp)


# speedup vs baseline: 23.5722x; 1.2701x over previous
import jax
import jax.numpy as jnp
from jax import lax
from jax.experimental import pallas as pl
from jax.experimental.pallas import tpu as pltpu
from jax.experimental.pallas import tpu_sc as plsc

N = 50000
E = 800000
NGRAPH = 64

NP = 50048
RB = 128
NBLK_TC = NP // RB
ETOT = E + N
EPAD = 851968
EPADV = EPAD - ETOT
EB = 128
NTILE = 16
NCORE = 2
R16 = NP // NTILE

_f32 = jnp.float32
_i32 = jnp.int32


def _sc_body_factory(hc, mode_a):
    def body(src_ref, dst_ref, htab_ref, atab_ref, zh_ref,
             acch_ref, acce_ref,
             sidx, didx, sidxt, didxt, hrows, ars, ard,
             acchs, sem):
        c = lax.axis_index("c")
        s = lax.axis_index("s")
        r0 = s * R16
        iota16 = lax.iota(_i32, 16)
        zeros16 = jnp.zeros((16,), _f32)

        if mode_a:
            nblk = EPAD // NTILE // EB
            e_base = s * (EPAD // NTILE)
            tab_off = c * NP
        else:
            nblk = EPAD // (NTILE * NCORE) // EB
            wid = c * NTILE + s
            e_base = wid * (EPAD // (NTILE * NCORE))
            tab_off = None

        def load_indices(base):
            ca = pltpu.async_copy(src_ref.at[pl.ds(base, EB)], sidx, sem)
            cb = pltpu.async_copy(dst_ref.at[pl.ds(base, EB)], didx, sem)
            ca.wait()
            cb.wait()
            if mode_a:
                for g in range(EB // 16):
                    sl = pl.ds(g * 16, 16)
                    sidxt[sl] = sidx[sl] + tab_off
                    didxt[sl] = didx[sl] + tab_off
                return sidxt, didxt
            return sidx, didx

        def edge_ex(g):
            eids = iota16 + g * 16
            exs = []
            for hh in range(hc):
                asv = plsc.load_gather(ars, [eids, jnp.full((16,), hh, _i32)])
                adv = plsc.load_gather(ard, [eids, jnp.full((16,), 8 + hh, _i32)])
                al = asv + adv
                al = jnp.where(al > 0.0, al, al * 0.2)
                exs.append(jnp.exp(al))
            return eids, exs

        @pl.loop(0, EB)
        def _zrow(r):
            hrows[r, pl.ds(0, 16)] = zeros16
            hrows[r, pl.ds(16, 16)] = zeros16

        pltpu.sync_copy(zh_ref.at[pl.ds(r0, R16)], acchs.at[pl.ds(r0, R16)])
        plsc.subcore_barrier()

        @pl.loop(0, nblk)
        def _denom_block(it):
            gsrc, gdst = load_indices(e_base + it * EB)
            g1 = pltpu.async_copy(atab_ref.at[gsrc], ars, sem)
            g2 = pltpu.async_copy(atab_ref.at[gdst], ard, sem)
            g1.wait()
            g2.wait()
            for g in range(EB // 16):
                eids, exs = edge_ex(g)
                for hh in range(hc):
                    plsc.store_scatter(hrows, [eids, jnp.full((16,), hh, _i32)],
                                       exs[hh])
            pltpu.sync_copy(hrows, acchs.at[didx], add=True)

        plsc.subcore_barrier()
        pltpu.sync_copy(acchs.at[pl.ds(r0, R16)],
                        acce_ref.at[c, pl.ds(r0, R16)])
        plsc.subcore_barrier()

        pltpu.sync_copy(zh_ref.at[pl.ds(r0, R16)], acchs.at[pl.ds(r0, R16)])
        plsc.subcore_barrier()

        @pl.loop(0, nblk)
        def _feat_block(it):
            gsrc, gdst = load_indices(e_base + it * EB)
            g0 = pltpu.async_copy(htab_ref.at[gsrc], hrows, sem)
            g1 = pltpu.async_copy(atab_ref.at[gsrc], ars, sem)
            g2 = pltpu.async_copy(atab_ref.at[gdst], ard, sem)
            g0.wait()
            g1.wait()
            g2.wait()
            for g in range(EB // 16):
                eids, exs = edge_ex(g)
                for col in range(32):
                    colv = jnp.full((16,), col, _i32)
                    hv = plsc.load_gather(hrows, [eids, colv])
                    plsc.store_scatter(hrows, [eids, colv],
                                       hv * exs[col * hc // 32])
            pltpu.sync_copy(hrows, acchs.at[didx], add=True)

        plsc.subcore_barrier()
        pltpu.sync_copy(acchs.at[pl.ds(r0, R16)],
                        acch_ref.at[c, pl.ds(r0, R16)])

    return body


def _make_sc_layer(hc, mode_a):
    mesh = plsc.VectorSubcoreMesh(core_axis_name="c", subcore_axis_name="s")
    return pl.kernel(
        _sc_body_factory(hc, mode_a),
        out_type=(jax.ShapeDtypeStruct((NCORE, NP, 32), _f32),
                  jax.ShapeDtypeStruct((NCORE, NP, 32), _f32)),
        mesh=mesh,
        scratch_types=[
            pltpu.VMEM((EB,), _i32),
            pltpu.VMEM((EB,), _i32),
            pltpu.VMEM((EB,), _i32),
            pltpu.VMEM((EB,), _i32),
            pltpu.VMEM((EB, 32), _f32),
            pltpu.VMEM((EB, 16), _f32),
            pltpu.VMEM((EB, 16), _f32),
            pltpu.VMEM_SHARED((NP, 32), _f32),
            pltpu.SemaphoreType.DMA,
        ],
        compiler_params=pltpu.CompilerParams(needs_layout_passes=False,
                                             use_tc_tiling_on_sc=False),
        name=f"gat_edge_hc{hc}_{'A' if mode_a else 'B'}",
    )


def _full(shape):
    return pl.BlockSpec(shape, lambda i: tuple(0 for _ in shape))


def _rows(width):
    return pl.BlockSpec((RB, width), lambda i: (i, 0))


def _chunk_tabs(width):
    return pl.BlockSpec((2, RB, width), lambda i: (0, i, 0))


def _tables_4h(h, asr, adt):
    htab = jnp.stack([h[:, :32], h[:, 32:]])
    z6 = jnp.zeros((RB, 6), _f32)
    a0 = jnp.concatenate([asr[:, 0:2], z6, adt[:, 0:2], z6], axis=1)
    a1 = jnp.concatenate([asr[:, 2:4], z6, adt[:, 2:4], z6], axis=1)
    return htab, jnp.stack([a0, a1])


def _k1_body(xf_ref, xb_ref, wf_ref, asf_ref, adf_ref,
             wb_ref, asb_ref, adb_ref,
             hf_ref, af_ref, hb_ref, ab_ref):
    for x_ref, w_ref, as_ref, ad_ref, h_out, a_out in (
            (xf_ref, wf_ref, asf_ref, adf_ref, hf_ref, af_ref),
            (xb_ref, wb_ref, asb_ref, adb_ref, hb_ref, ab_ref)):
        x = x_ref[...]
        h = jnp.dot(x, w_ref[...], preferred_element_type=_f32)
        asr = jnp.dot(h, as_ref[...], preferred_element_type=_f32)
        adt = jnp.dot(h, ad_ref[...], preferred_element_type=_f32)
        htab, atab = _tables_4h(h, asr, adt)
        h_out[...] = htab
        a_out[...] = atab


def _combine_4h(acch, acce, bias):
    cols = []
    for c in range(2):
        for hh in range(2):
            num = acch[c][:, hh * 16:(hh + 1) * 16]
            den = acce[c][:, hh:hh + 1] + 1e-16
            cols.append(num / den)
    v = jnp.concatenate(cols, axis=1) + bias
    return jnp.where(v > 0.0, v, jnp.exp(v) - 1.0)


def _k2_body(ahf_ref, aef_ref, ahb_ref, aeb_ref,
             bf_ref, wf_ref, asf_ref, adf_ref,
             bb_ref, wb_ref, asb_ref, adb_ref,
             hf_ref, af_ref, hb_ref, ab_ref):
    for ah_ref, ae_ref, b_ref, w_ref, as_ref, ad_ref, h_out, a_out in (
            (ahf_ref, aef_ref, bf_ref, wf_ref, asf_ref, adf_ref, hf_ref, af_ref),
            (ahb_ref, aeb_ref, bb_ref, wb_ref, asb_ref, adb_ref, hb_ref, ab_ref)):
        xin = _combine_4h(ah_ref[...], ae_ref[...], b_ref[...])
        h = jnp.dot(xin, w_ref[...], preferred_element_type=_f32)
        asr = jnp.dot(h, as_ref[...], preferred_element_type=_f32)
        adt = jnp.dot(h, ad_ref[...], preferred_element_type=_f32)
        htab, atab = _tables_4h(h, asr, adt)
        h_out[...] = htab
        a_out[...] = atab


def _k3_body(ahf_ref, aef_ref, ahb_ref, aeb_ref,
             bf_ref, wf_ref, asf_ref, adf_ref,
             bb_ref, wb_ref, asb_ref, adb_ref,
             hf_ref, af_ref, hb_ref, ab_ref):
    z7 = jnp.zeros((RB, 7), _f32)
    for ah_ref, ae_ref, b_ref, w_ref, as_ref, ad_ref, h_out, a_out in (
            (ahf_ref, aef_ref, bf_ref, wf_ref, asf_ref, adf_ref, hf_ref, af_ref),
            (ahb_ref, aeb_ref, bb_ref, wb_ref, asb_ref, adb_ref, hb_ref, ab_ref)):
        xin = _combine_4h(ah_ref[...], ae_ref[...], b_ref[...])
        h = jnp.dot(xin, w_ref[...], preferred_element_type=_f32)
        asr = jnp.dot(h, as_ref[...], preferred_element_type=_f32)
        adt = jnp.dot(h, ad_ref[...], preferred_element_type=_f32)
        h_out[...] = h
        a_out[...] = jnp.concatenate([asr, z7, adt, z7], axis=1)


def _kpool_body(ahf_ref, aef_ref, ahb_ref, aeb_ref,
                bf_ref, bb_ref, batch_ref,
                hnode_ref, gpool_ref, accp, accc):
    i = pl.program_id(0)
    parts = []
    for ah_ref, ae_ref, b_ref in ((ahf_ref, aef_ref, bf_ref),
                                  (ahb_ref, aeb_ref, bb_ref)):
        ah = ah_ref[...]
        ae = ae_ref[...]
        num = ah[0] + ah[1]
        den = ae[0][:, 0:1] + ae[1][:, 0:1] + 1e-16
        parts.append(num / den + b_ref[...])
    hn = jnp.concatenate(parts, axis=1)
    hnode_ref[...] = hn

    row = lax.broadcasted_iota(_i32, (RB, 1), 0) + i * RB
    valid = row < N
    bt = batch_ref[0, 0, :].reshape(RB, 1)
    gid = lax.broadcasted_iota(_i32, (RB, NGRAPH), 1)
    oh = jnp.where((bt == gid) & valid, 1.0, 0.0).astype(_f32)
    contrib = lax.dot_general(oh, hn, (((0,), (0,)), ((), ())),
                              preferred_element_type=_f32)
    cnt = jnp.sum(oh, axis=0).reshape(NGRAPH, 1)
    newp = jnp.where(i == 0, contrib, accp[...] + contrib)
    newc = jnp.where(i == 0, cnt, accc[...] + cnt)
    accp[...] = newp
    accc[...] = newc

    @pl.when(i == NBLK_TC - 1)
    def _():
        gpool_ref[...] = newp / jnp.clip(newc, 1.0)


def _run(xf, xb, src, dst, batchp, pp):
    zh = jnp.zeros((NP, 32), _f32)

    k1 = pl.pallas_call(
        _k1_body,
        grid=(NBLK_TC,),
        in_specs=[_rows(8), _rows(8),
                  _full((8, 64)), _full((64, 4)), _full((64, 4)),
                  _full((8, 64)), _full((64, 4)), _full((64, 4))],
        out_specs=[_chunk_tabs(32), _chunk_tabs(16),
                   _chunk_tabs(32), _chunk_tabs(16)],
        out_shape=[jax.ShapeDtypeStruct((2, NP, 32), _f32),
                   jax.ShapeDtypeStruct((2, NP, 16), _f32),
                   jax.ShapeDtypeStruct((2, NP, 32), _f32),
                   jax.ShapeDtypeStruct((2, NP, 16), _f32)],
    )
    hf1, af1, hb1, ab1 = k1(xf, xb, pp['f1W'], pp['f1As'], pp['f1Ad'],
                            pp['b1W'], pp['b1As'], pp['b1Ad'])

    sc4 = _make_sc_layer(2, True)
    sc1 = _make_sc_layer(1, False)

    ahf1, aef1 = sc4(src, dst, hf1.reshape(2 * NP, 32),
                     af1.reshape(2 * NP, 16), zh)
    ahb1, aeb1 = sc4(dst, src, hb1.reshape(2 * NP, 32),
                     ab1.reshape(2 * NP, 16), zh)

    k2 = pl.pallas_call(
        _k2_body,
        grid=(NBLK_TC,),
        in_specs=[_chunk_tabs(32), _chunk_tabs(32),
                  _chunk_tabs(32), _chunk_tabs(32),
                  _full((1, 64)), _full((64, 64)), _full((64, 4)), _full((64, 4)),
                  _full((1, 64)), _full((64, 64)), _full((64, 4)), _full((64, 4))],
        out_specs=[_chunk_tabs(32), _chunk_tabs(16),
                   _chunk_tabs(32), _chunk_tabs(16)],
        out_shape=[jax.ShapeDtypeStruct((2, NP, 32), _f32),
                   jax.ShapeDtypeStruct((2, NP, 16), _f32),
                   jax.ShapeDtypeStruct((2, NP, 32), _f32),
                   jax.ShapeDtypeStruct((2, NP, 16), _f32)],
    )
    hf2, af2, hb2, ab2 = k2(ahf1, aef1, ahb1, aeb1,
                            pp['f1b'], pp['f2W'], pp['f2As'], pp['f2Ad'],
                            pp['b1b'], pp['b2W'], pp['b2As'], pp['b2Ad'])

    ahf2, aef2 = sc4(src, dst, hf2.reshape(2 * NP, 32),
                     af2.reshape(2 * NP, 16), zh)
    ahb2, aeb2 = sc4(dst, src, hb2.reshape(2 * NP, 32),
                     ab2.reshape(2 * NP, 16), zh)

    k3 = pl.pallas_call(
        _k3_body,
        grid=(NBLK_TC,),
        in_specs=[_chunk_tabs(32), _chunk_tabs(32),
                  _chunk_tabs(32), _chunk_tabs(32),
                  _full((1, 64)), _full((64, 32)), _full((32, 1)), _full((32, 1)),
                  _full((1, 64)), _full((64, 32)), _full((32, 1)), _full((32, 1))],
        out_specs=[_rows(32), _rows(16), _rows(32), _rows(16)],
        out_shape=[jax.ShapeDtypeStruct((NP, 32), _f32),
                   jax.ShapeDtypeStruct((NP, 16), _f32),
                   jax.ShapeDtypeStruct((NP, 32), _f32),
                   jax.ShapeDtypeStruct((NP, 16), _f32)],
    )
    hf3, af3, hb3, ab3 = k3(ahf2, aef2, ahb2, aeb2,
                            pp['f2b'], pp['f3W'], pp['f3As'], pp['f3Ad'],
                            pp['b2b'], pp['b3W'], pp['b3As'], pp['b3Ad'])

    ahf3, aef3 = sc1(src, dst, hf3, af3, zh)
    ahb3, aeb3 = sc1(dst, src, hb3, ab3, zh)

    kpool = pl.pallas_call(
        _kpool_body,
        grid=(NBLK_TC,),
        in_specs=[_chunk_tabs(32), _chunk_tabs(32),
                  _chunk_tabs(32), _chunk_tabs(32),
                  _full((1, 32)), _full((1, 32)),
                  pl.BlockSpec((1, 1, RB), lambda i: (i, 0, 0))],
        out_specs=[_rows(64), pl.BlockSpec((NGRAPH, NGRAPH), lambda i: (0, 0))],
        out_shape=[jax.ShapeDtypeStruct((N, 64), _f32),
                   jax.ShapeDtypeStruct((NGRAPH, NGRAPH), _f32)],
        scratch_shapes=[pltpu.VMEM((NGRAPH, NGRAPH), _f32),
                        pltpu.VMEM((NGRAPH, 1), _f32)],
    )
    h_node, g_pool = kpool(ahf3, aef3, ahb3, aeb3,
                           pp['f3b'], pp['b3b'], batchp)
    return h_node, g_pool


def _attn_mat(a, heads, out_ch):
    if heads == 1:
        return a.reshape(out_ch, 1)
    eye = jnp.repeat(jnp.eye(heads, dtype=_f32), out_ch, axis=0)
    return a.reshape(heads * out_ch, 1) * eye


def kernel(x, params, edge_index, batch):
    ei = edge_index.astype(_i32)
    loops = jnp.arange(N, dtype=_i32)
    padv = jnp.full((EPADV,), N, _i32)
    src = jnp.concatenate([ei[0], loops, padv])
    dst = jnp.concatenate([ei[1], loops, padv])

    xf = jnp.pad(x[:, jnp.array([0, 1, 3])], ((0, NP - N), (0, 5)))
    xb = jnp.pad(x[:, jnp.array([0, 2, 4])], ((0, NP - N), (0, 5)))
    batchp = jnp.pad(batch.astype(_i32), (0, NP - N)).reshape(NBLK_TC, 1, RB)

    pp = {}
    for name, heads, out_ch in (('f1', 4, 16), ('f2', 4, 16), ('f3', 1, 32),
                                ('b1', 4, 16), ('b2', 4, 16), ('b3', 1, 32)):
        p = params[name]
        w = p['W']
        if w.shape[0] == 3:
            w = jnp.pad(w, ((0, 5), (0, 0)))
        pp[name + 'W'] = w
        pp[name + 'As'] = _attn_mat(p['a_s'], heads, out_ch)
        pp[name + 'Ad'] = _attn_mat(p['a_d'], heads, out_ch)
        pp[name + 'b'] = p['b'].reshape(1, -1)

    return _run(xf, xb, src, dst, batchp, pp)


# speedup vs baseline: 26.3058x; 1.1160x over previous
import jax
import jax.numpy as jnp
from jax import lax
from jax.experimental import pallas as pl
from jax.experimental.pallas import tpu as pltpu
from jax.experimental.pallas import tpu_sc as plsc

N = 50000
E = 800000
NGRAPH = 64

NP = 50048
RB = 128
NBLK_TC = NP // RB
ETOT = E + N
EPAD = 851968
EB = 128
EPADV = EPAD + EB - ETOT
NTILE = 16
NCORE = 2
R16 = NP // NTILE

_f32 = jnp.float32
_i32 = jnp.int32


def _sc_body_factory(hc, mode_a):
    def body(src_ref, dst_ref, htab_ref, atab_ref, zh_ref,
             acch_ref, acce_ref,
             sidx0, sidx1, didx0, didx1, sidxt0, sidxt1, didxt0, didxt1,
             hrows0, hrows1, ars0, ars1, ard0, ard1,
             acchs, semi, semg0, semg1, sems0, sems1):
        sidx = (sidx0, sidx1)
        didx = (didx0, didx1)
        sidxt = (sidxt0, sidxt1)
        didxt = (didxt0, didxt1)
        hrows = (hrows0, hrows1)
        ars = (ars0, ars1)
        ard = (ard0, ard1)
        semg = (semg0, semg1)
        sems = (sems0, sems1)

        c = lax.axis_index("c")
        s = lax.axis_index("s")
        r0 = s * R16
        iota16 = lax.iota(_i32, 16)
        zeros16 = jnp.zeros((16,), _f32)

        if mode_a:
            nblk = EPAD // NTILE // EB
            e_base = s * (EPAD // NTILE)
            tab_off = c * NP
        else:
            nblk = EPAD // (NTILE * NCORE) // EB
            wid = c * NTILE + s
            e_base = wid * (EPAD // (NTILE * NCORE))
            tab_off = None

        def idx_issue(ib, nb):
            base = e_base + ib * EB
            ci = pltpu.async_copy(src_ref.at[pl.ds(base, EB)], sidx[nb], semi)
            cj = pltpu.async_copy(dst_ref.at[pl.ds(base, EB)], didx[nb], semi)
            return ci, cj

        def idx_finish(ci, cj, nb):
            ci.wait()
            cj.wait()
            if mode_a:
                for g in range(EB // 16):
                    sl = pl.ds(g * 16, 16)
                    sidxt[nb][sl] = sidx[nb][sl] + tab_off
                    didxt[nb][sl] = didx[nb][sl] + tab_off

        def gsrc(b):
            return sidxt[b] if mode_a else sidx[b]

        def gdst(b):
            return didxt[b] if mode_a else didx[b]

        def gathers_issue(b, with_h):
            pltpu.async_copy(atab_ref.at[gsrc(b)], ars[b], semg[b])
            pltpu.async_copy(atab_ref.at[gdst(b)], ard[b], semg[b])
            if with_h:
                pltpu.async_copy(htab_ref.at[gsrc(b)], hrows[b], semg[b])

        def gathers_wait(b, with_h):
            pltpu.make_async_copy(atab_ref.at[gsrc(b)], ars[b], semg[b]).wait()
            pltpu.make_async_copy(atab_ref.at[gdst(b)], ard[b], semg[b]).wait()
            if with_h:
                pltpu.make_async_copy(htab_ref.at[gsrc(b)], hrows[b],
                                      semg[b]).wait()

        def scatter_issue(b):
            pltpu.async_copy(hrows[b], acchs.at[didx[b]], sems[b], add=True)

        def scatter_wait(b):
            pltpu.make_async_copy(hrows[b], acchs.at[didx[b]], sems[b]).wait()

        def edge_ex(g, b):
            eids = iota16 + g * 16
            exs = []
            for hh in range(hc):
                asv = plsc.load_gather(ars[b],
                                       [eids, jnp.full((16,), hh, _i32)])
                adv = plsc.load_gather(ard[b],
                                       [eids, jnp.full((16,), 8 + hh, _i32)])
                al = asv + adv
                al = jnp.where(al > 0.0, al, al * 0.2)
                exs.append(jnp.exp(al))
            return eids, exs

        def compute_denom(b):
            for g in range(EB // 16):
                eids, exs = edge_ex(g, b)
                for hh in range(hc):
                    plsc.store_scatter(hrows[b],
                                       [eids, jnp.full((16,), hh, _i32)],
                                       exs[hh])

        def compute_feat(b):
            for g in range(EB // 16):
                eids, exs = edge_ex(g, b)
                for col in range(32):
                    colv = jnp.full((16,), col, _i32)
                    hv = plsc.load_gather(hrows[b], [eids, colv])
                    plsc.store_scatter(hrows[b], [eids, colv],
                                       hv * exs[col * hc // 32])

        def run_phase(with_h, compute):
            ci, cj = idx_issue(0, 0)
            idx_finish(ci, cj, 0)
            gathers_issue(0, with_h)

            def step(ib, b, first):
                nb = 1 - b
                gathers_wait(b, with_h)
                if not first:
                    scatter_wait(nb)
                ci, cj = idx_issue(ib + 1, nb)
                compute(b)
                scatter_issue(b)
                idx_finish(ci, cj, nb)
                gathers_issue(nb, with_h)

            step(0, 0, True)
            step(1, 1, False)

            @pl.loop(2, nblk, step=2)
            def _pipe(it):
                step(it, 0, False)
                step(it + 1, 1, False)

            gathers_wait(0, with_h)
            scatter_wait(1)

        @pl.loop(0, EB)
        def _zrow(r):
            for b in range(2):
                hrows[b][r, pl.ds(0, 16)] = zeros16
                hrows[b][r, pl.ds(16, 16)] = zeros16

        pltpu.sync_copy(zh_ref.at[pl.ds(r0, R16)], acchs.at[pl.ds(r0, R16)])
        plsc.subcore_barrier()
        run_phase(False, compute_denom)
        plsc.subcore_barrier()
        pltpu.sync_copy(acchs.at[pl.ds(r0, R16)],
                        acce_ref.at[c, pl.ds(r0, R16)])
        plsc.subcore_barrier()

        pltpu.sync_copy(zh_ref.at[pl.ds(r0, R16)], acchs.at[pl.ds(r0, R16)])
        plsc.subcore_barrier()
        run_phase(True, compute_feat)
        plsc.subcore_barrier()
        pltpu.sync_copy(acchs.at[pl.ds(r0, R16)],
                        acch_ref.at[c, pl.ds(r0, R16)])

    return body


def _make_sc_layer(hc, mode_a):
    mesh = plsc.VectorSubcoreMesh(core_axis_name="c", subcore_axis_name="s")
    return pl.kernel(
        _sc_body_factory(hc, mode_a),
        out_type=(jax.ShapeDtypeStruct((NCORE, NP, 32), _f32),
                  jax.ShapeDtypeStruct((NCORE, NP, 32), _f32)),
        mesh=mesh,
        scratch_types=(
            [pltpu.VMEM((EB,), _i32)] * 8 +
            [pltpu.VMEM((EB, 32), _f32)] * 2 +
            [pltpu.VMEM((EB, 16), _f32)] * 4 +
            [pltpu.VMEM_SHARED((NP, 32), _f32)] +
            [pltpu.SemaphoreType.DMA] * 5
        ),
        compiler_params=pltpu.CompilerParams(needs_layout_passes=False,
                                             use_tc_tiling_on_sc=False),
        name=f"gat_edge_hc{hc}_{'A' if mode_a else 'B'}",
    )


def _full(shape):
    return pl.BlockSpec(shape, lambda i: tuple(0 for _ in shape))


def _rows(width):
    return pl.BlockSpec((RB, width), lambda i: (i, 0))


def _chunk_tabs(width):
    return pl.BlockSpec((2, RB, width), lambda i: (0, i, 0))


def _tables_4h(h, asr, adt):
    htab = jnp.stack([h[:, :32], h[:, 32:]])
    z6 = jnp.zeros((RB, 6), _f32)
    a0 = jnp.concatenate([asr[:, 0:2], z6, adt[:, 0:2], z6], axis=1)
    a1 = jnp.concatenate([asr[:, 2:4], z6, adt[:, 2:4], z6], axis=1)
    return htab, jnp.stack([a0, a1])


def _k1_body(xf_ref, xb_ref, wf_ref, asf_ref, adf_ref,
             wb_ref, asb_ref, adb_ref,
             hf_ref, af_ref, hb_ref, ab_ref):
    for x_ref, w_ref, as_ref, ad_ref, h_out, a_out in (
            (xf_ref, wf_ref, asf_ref, adf_ref, hf_ref, af_ref),
            (xb_ref, wb_ref, asb_ref, adb_ref, hb_ref, ab_ref)):
        x = x_ref[...]
        h = jnp.dot(x, w_ref[...], preferred_element_type=_f32)
        asr = jnp.dot(h, as_ref[...], preferred_element_type=_f32)
        adt = jnp.dot(h, ad_ref[...], preferred_element_type=_f32)
        htab, atab = _tables_4h(h, asr, adt)
        h_out[...] = htab
        a_out[...] = atab


def _combine_4h(acch, acce, bias):
    cols = []
    for c in range(2):
        for hh in range(2):
            num = acch[c][:, hh * 16:(hh + 1) * 16]
            den = acce[c][:, hh:hh + 1] + 1e-16
            cols.append(num / den)
    v = jnp.concatenate(cols, axis=1) + bias
    return jnp.where(v > 0.0, v, jnp.exp(v) - 1.0)


def _k2_body(ahf_ref, aef_ref, ahb_ref, aeb_ref,
             bf_ref, wf_ref, asf_ref, adf_ref,
             bb_ref, wb_ref, asb_ref, adb_ref,
             hf_ref, af_ref, hb_ref, ab_ref):
    for ah_ref, ae_ref, b_ref, w_ref, as_ref, ad_ref, h_out, a_out in (
            (ahf_ref, aef_ref, bf_ref, wf_ref, asf_ref, adf_ref, hf_ref, af_ref),
            (ahb_ref, aeb_ref, bb_ref, wb_ref, asb_ref, adb_ref, hb_ref, ab_ref)):
        xin = _combine_4h(ah_ref[...], ae_ref[...], b_ref[...])
        h = jnp.dot(xin, w_ref[...], preferred_element_type=_f32)
        asr = jnp.dot(h, as_ref[...], preferred_element_type=_f32)
        adt = jnp.dot(h, ad_ref[...], preferred_element_type=_f32)
        htab, atab = _tables_4h(h, asr, adt)
        h_out[...] = htab
        a_out[...] = atab


def _k3_body(ahf_ref, aef_ref, ahb_ref, aeb_ref,
             bf_ref, wf_ref, asf_ref, adf_ref,
             bb_ref, wb_ref, asb_ref, adb_ref,
             hf_ref, af_ref, hb_ref, ab_ref):
    z7 = jnp.zeros((RB, 7), _f32)
    for ah_ref, ae_ref, b_ref, w_ref, as_ref, ad_ref, h_out, a_out in (
            (ahf_ref, aef_ref, bf_ref, wf_ref, asf_ref, adf_ref, hf_ref, af_ref),
            (ahb_ref, aeb_ref, bb_ref, wb_ref, asb_ref, adb_ref, hb_ref, ab_ref)):
        xin = _combine_4h(ah_ref[...], ae_ref[...], b_ref[...])
        h = jnp.dot(xin, w_ref[...], preferred_element_type=_f32)
        asr = jnp.dot(h, as_ref[...], preferred_element_type=_f32)
        adt = jnp.dot(h, ad_ref[...], preferred_element_type=_f32)
        h_out[...] = h
        a_out[...] = jnp.concatenate([asr, z7, adt, z7], axis=1)


def _kpool_body(ahf_ref, aef_ref, ahb_ref, aeb_ref,
                bf_ref, bb_ref, batch_ref,
                hnode_ref, gpool_ref, accp, accc):
    i = pl.program_id(0)
    parts = []
    for ah_ref, ae_ref, b_ref in ((ahf_ref, aef_ref, bf_ref),
                                  (ahb_ref, aeb_ref, bb_ref)):
        ah = ah_ref[...]
        ae = ae_ref[...]
        num = ah[0] + ah[1]
        den = ae[0][:, 0:1] + ae[1][:, 0:1] + 1e-16
        parts.append(num / den + b_ref[...])
    hn = jnp.concatenate(parts, axis=1)
    hnode_ref[...] = hn

    row = lax.broadcasted_iota(_i32, (RB, 1), 0) + i * RB
    valid = row < N
    bt = batch_ref[0, 0, :].reshape(RB, 1)
    gid = lax.broadcasted_iota(_i32, (RB, NGRAPH), 1)
    oh = jnp.where((bt == gid) & valid, 1.0, 0.0).astype(_f32)
    contrib = lax.dot_general(oh, hn, (((0,), (0,)), ((), ())),
                              preferred_element_type=_f32)
    cnt = jnp.sum(oh, axis=0).reshape(NGRAPH, 1)
    newp = jnp.where(i == 0, contrib, accp[...] + contrib)
    newc = jnp.where(i == 0, cnt, accc[...] + cnt)
    accp[...] = newp
    accc[...] = newc

    @pl.when(i == NBLK_TC - 1)
    def _():
        gpool_ref[...] = newp / jnp.clip(newc, 1.0)


def _run(xf, xb, src, dst, batchp, pp):
    zh = jnp.zeros((NP, 32), _f32)

    k1 = pl.pallas_call(
        _k1_body,
        grid=(NBLK_TC,),
        in_specs=[_rows(8), _rows(8),
                  _full((8, 64)), _full((64, 4)), _full((64, 4)),
                  _full((8, 64)), _full((64, 4)), _full((64, 4))],
        out_specs=[_chunk_tabs(32), _chunk_tabs(16),
                   _chunk_tabs(32), _chunk_tabs(16)],
        out_shape=[jax.ShapeDtypeStruct((2, NP, 32), _f32),
                   jax.ShapeDtypeStruct((2, NP, 16), _f32),
                   jax.ShapeDtypeStruct((2, NP, 32), _f32),
                   jax.ShapeDtypeStruct((2, NP, 16), _f32)],
    )
    hf1, af1, hb1, ab1 = k1(xf, xb, pp['f1W'], pp['f1As'], pp['f1Ad'],
                            pp['b1W'], pp['b1As'], pp['b1Ad'])

    sc4 = _make_sc_layer(2, True)
    sc1 = _make_sc_layer(1, False)

    ahf1, aef1 = sc4(src, dst, hf1.reshape(2 * NP, 32),
                     af1.reshape(2 * NP, 16), zh)
    ahb1, aeb1 = sc4(dst, src, hb1.reshape(2 * NP, 32),
                     ab1.reshape(2 * NP, 16), zh)

    k2 = pl.pallas_call(
        _k2_body,
        grid=(NBLK_TC,),
        in_specs=[_chunk_tabs(32), _chunk_tabs(32),
                  _chunk_tabs(32), _chunk_tabs(32),
                  _full((1, 64)), _full((64, 64)), _full((64, 4)), _full((64, 4)),
                  _full((1, 64)), _full((64, 64)), _full((64, 4)), _full((64, 4))],
        out_specs=[_chunk_tabs(32), _chunk_tabs(16),
                   _chunk_tabs(32), _chunk_tabs(16)],
        out_shape=[jax.ShapeDtypeStruct((2, NP, 32), _f32),
                   jax.ShapeDtypeStruct((2, NP, 16), _f32),
                   jax.ShapeDtypeStruct((2, NP, 32), _f32),
                   jax.ShapeDtypeStruct((2, NP, 16), _f32)],
    )
    hf2, af2, hb2, ab2 = k2(ahf1, aef1, ahb1, aeb1,
                            pp['f1b'], pp['f2W'], pp['f2As'], pp['f2Ad'],
                            pp['b1b'], pp['b2W'], pp['b2As'], pp['b2Ad'])

    ahf2, aef2 = sc4(src, dst, hf2.reshape(2 * NP, 32),
                     af2.reshape(2 * NP, 16), zh)
    ahb2, aeb2 = sc4(dst, src, hb2.reshape(2 * NP, 32),
                     ab2.reshape(2 * NP, 16), zh)

    k3 = pl.pallas_call(
        _k3_body,
        grid=(NBLK_TC,),
        in_specs=[_chunk_tabs(32), _chunk_tabs(32),
                  _chunk_tabs(32), _chunk_tabs(32),
                  _full((1, 64)), _full((64, 32)), _full((32, 1)), _full((32, 1)),
                  _full((1, 64)), _full((64, 32)), _full((32, 1)), _full((32, 1))],
        out_specs=[_rows(32), _rows(16), _rows(32), _rows(16)],
        out_shape=[jax.ShapeDtypeStruct((NP, 32), _f32),
                   jax.ShapeDtypeStruct((NP, 16), _f32),
                   jax.ShapeDtypeStruct((NP, 32), _f32),
                   jax.ShapeDtypeStruct((NP, 16), _f32)],
    )
    hf3, af3, hb3, ab3 = k3(ahf2, aef2, ahb2, aeb2,
                            pp['f2b'], pp['f3W'], pp['f3As'], pp['f3Ad'],
                            pp['b2b'], pp['b3W'], pp['b3As'], pp['b3Ad'])

    ahf3, aef3 = sc1(src, dst, hf3, af3, zh)
    ahb3, aeb3 = sc1(dst, src, hb3, ab3, zh)

    kpool = pl.pallas_call(
        _kpool_body,
        grid=(NBLK_TC,),
        in_specs=[_chunk_tabs(32), _chunk_tabs(32),
                  _chunk_tabs(32), _chunk_tabs(32),
                  _full((1, 32)), _full((1, 32)),
                  pl.BlockSpec((1, 1, RB), lambda i: (i, 0, 0))],
        out_specs=[_rows(64), pl.BlockSpec((NGRAPH, NGRAPH), lambda i: (0, 0))],
        out_shape=[jax.ShapeDtypeStruct((N, 64), _f32),
                   jax.ShapeDtypeStruct((NGRAPH, NGRAPH), _f32)],
        scratch_shapes=[pltpu.VMEM((NGRAPH, NGRAPH), _f32),
                        pltpu.VMEM((NGRAPH, 1), _f32)],
    )
    h_node, g_pool = kpool(ahf3, aef3, ahb3, aeb3,
                           pp['f3b'], pp['b3b'], batchp)
    return h_node, g_pool


def _attn_mat(a, heads, out_ch):
    if heads == 1:
        return a.reshape(out_ch, 1)
    eye = jnp.repeat(jnp.eye(heads, dtype=_f32), out_ch, axis=0)
    return a.reshape(heads * out_ch, 1) * eye


def kernel(x, params, edge_index, batch):
    ei = edge_index.astype(_i32)
    loops = jnp.arange(N, dtype=_i32)
    padv = jnp.full((EPADV,), N, _i32)
    src = jnp.concatenate([ei[0], loops, padv])
    dst = jnp.concatenate([ei[1], loops, padv])

    xf = jnp.pad(x[:, jnp.array([0, 1, 3])], ((0, NP - N), (0, 5)))
    xb = jnp.pad(x[:, jnp.array([0, 2, 4])], ((0, NP - N), (0, 5)))
    batchp = jnp.pad(batch.astype(_i32), (0, NP - N)).reshape(NBLK_TC, 1, RB)

    pp = {}
    for name, heads, out_ch in (('f1', 4, 16), ('f2', 4, 16), ('f3', 1, 32),
                                ('b1', 4, 16), ('b2', 4, 16), ('b3', 1, 32)):
        p = params[name]
        w = p['W']
        if w.shape[0] == 3:
            w = jnp.pad(w, ((0, 5), (0, 0)))
        pp[name + 'W'] = w
        pp[name + 'As'] = _attn_mat(p['a_s'], heads, out_ch)
        pp[name + 'Ad'] = _attn_mat(p['a_d'], heads, out_ch)
        pp[name + 'b'] = p['b'].reshape(1, -1)

    return _run(xf, xb, src, dst, batchp, pp)


# speedup vs baseline: 36.8477x; 1.4007x over previous
import jax
import jax.numpy as jnp
from jax import lax
from jax.experimental import pallas as pl
from jax.experimental.pallas import tpu as pltpu
from jax.experimental.pallas import tpu_sc as plsc

N = 50000
E = 800000
NGRAPH = 64

NP = 50048
RB = 128
NBLK_TC = NP // RB
ETOT = E + N
EPAD = 851968
EB = 128
EPADV = EPAD + EB - ETOT
NTILE = 16
NCORE = 2
R16 = NP // NTILE

_f32 = jnp.float32
_i32 = jnp.int32


def _sc_body_factory(hc, mode_a):
    def body(src_ref, dst_ref, htaba_ref, htabb_ref, atab_ref, zh_ref,
             acch_ref, acce_ref,
             sidx0, sidx1, didx0, didx1, sidxt0, sidxt1, didxt0, didxt1,
             d2a0, d2a1, d2b0, d2b1,
             hra0, hra1, hrb0, hrb1, ars0, ars1, ard0, ard1,
             acchs, semi, semg0, semg1, sems0, sems1):
        sidx = (sidx0, sidx1)
        didx = (didx0, didx1)
        sidxt = (sidxt0, sidxt1)
        didxt = (didxt0, didxt1)
        d2a = (d2a0, d2a1)
        d2b = (d2b0, d2b1)
        hra = (hra0, hra1)
        hrb = (hrb0, hrb1)
        ars = (ars0, ars1)
        ard = (ard0, ard1)
        semg = (semg0, semg1)
        sems = (sems0, sems1)

        c = lax.axis_index("c")
        s = lax.axis_index("s")
        r0 = s * R16
        iota16 = lax.iota(_i32, 16)
        zeros16 = jnp.zeros((16,), _f32)

        if mode_a:
            nblk = EPAD // NTILE // EB
            e_base = s * (EPAD // NTILE)
            tab_off = c * NP
        else:
            nblk = EPAD // (NTILE * NCORE) // EB
            wid = c * NTILE + s
            e_base = wid * (EPAD // (NTILE * NCORE))
            tab_off = None

        def idx_issue(ib, nb):
            base = e_base + ib * EB
            ci = pltpu.async_copy(src_ref.at[pl.ds(base, EB)], sidx[nb], semi)
            cj = pltpu.async_copy(dst_ref.at[pl.ds(base, EB)], didx[nb], semi)
            return ci, cj

        def idx_finish(ci, cj, nb):
            ci.wait()
            cj.wait()
            for g in range(EB // 16):
                sl = pl.ds(g * 16, 16)
                dv = didx[nb][sl]
                d2a[nb][sl] = dv * 2
                d2b[nb][sl] = dv * 2 + 1
                if mode_a:
                    sidxt[nb][sl] = sidx[nb][sl] + tab_off
                    didxt[nb][sl] = dv + tab_off

        def gsrc(b):
            return sidxt[b] if mode_a else sidx[b]

        def gdst(b):
            return didxt[b] if mode_a else didx[b]

        def gathers_issue(b, with_h):
            pltpu.async_copy(atab_ref.at[gsrc(b)], ars[b], semg[b])
            pltpu.async_copy(atab_ref.at[gdst(b)], ard[b], semg[b])
            if with_h:
                pltpu.async_copy(htaba_ref.at[gsrc(b)], hra[b], semg[b])
                pltpu.async_copy(htabb_ref.at[gsrc(b)], hrb[b], semg[b])

        def gathers_wait(b, with_h):
            pltpu.make_async_copy(atab_ref.at[gsrc(b)], ars[b], semg[b]).wait()
            pltpu.make_async_copy(atab_ref.at[gdst(b)], ard[b], semg[b]).wait()
            if with_h:
                pltpu.make_async_copy(htaba_ref.at[gsrc(b)], hra[b],
                                      semg[b]).wait()
                pltpu.make_async_copy(htabb_ref.at[gsrc(b)], hrb[b],
                                      semg[b]).wait()

        def scatter_issue(b, with_h):
            pltpu.async_copy(hra[b], acchs.at[d2a[b]], sems[b], add=True)
            if with_h:
                pltpu.async_copy(hrb[b], acchs.at[d2b[b]], sems[b], add=True)

        def scatter_wait(b, with_h):
            pltpu.make_async_copy(hra[b], acchs.at[d2a[b]], sems[b]).wait()
            if with_h:
                pltpu.make_async_copy(hrb[b], acchs.at[d2b[b]],
                                      sems[b]).wait()

        def edge_ex(g, b):
            eids = iota16 + g * 16
            exs = []
            for hh in range(hc):
                asv = plsc.load_gather(ars[b],
                                       [eids, jnp.full((16,), hh, _i32)])
                adv = plsc.load_gather(ard[b],
                                       [eids, jnp.full((16,), 8 + hh, _i32)])
                al = asv + adv
                al = jnp.where(al > 0.0, al, al * 0.2)
                exs.append(jnp.exp(al))
            return eids, exs

        def compute_denom(b):
            for g in range(EB // 16):
                eids, exs = edge_ex(g, b)
                for hh in range(hc):
                    plsc.store_scatter(hra[b],
                                       [eids, jnp.full((16,), hh, _i32)],
                                       exs[hh])

        def compute_feat(b):
            for g in range(EB // 16):
                eids, exs = edge_ex(g, b)
                for col in range(32):
                    buf = hra[b] if col < 16 else hrb[b]
                    colv = jnp.full((16,), col % 16, _i32)
                    hv = plsc.load_gather(buf, [eids, colv])
                    plsc.store_scatter(buf, [eids, colv],
                                       hv * exs[col * hc // 32])

        def run_phase(with_h, compute):
            ci, cj = idx_issue(0, 0)
            idx_finish(ci, cj, 0)
            gathers_issue(0, with_h)

            def step(ib, b, first):
                nb = 1 - b
                gathers_wait(b, with_h)
                if not first:
                    scatter_wait(nb, with_h)
                ci, cj = idx_issue(ib + 1, nb)
                compute(b)
                scatter_issue(b, with_h)
                idx_finish(ci, cj, nb)
                gathers_issue(nb, with_h)

            step(0, 0, True)
            step(1, 1, False)

            @pl.loop(2, nblk, step=2)
            def _pipe(it):
                step(it, 0, False)
                step(it + 1, 1, False)

            gathers_wait(0, with_h)
            scatter_wait(1, with_h)

        @pl.loop(0, EB)
        def _zrow(r):
            for b in range(2):
                hra[b][r, pl.ds(0, 16)] = zeros16

        pltpu.sync_copy(zh_ref.at[pl.ds(2 * r0, 2 * R16)],
                        acchs.at[pl.ds(2 * r0, 2 * R16)])
        plsc.subcore_barrier()
        run_phase(False, compute_denom)
        plsc.subcore_barrier()
        pltpu.sync_copy(acchs.at[pl.ds(2 * r0, 2 * R16)],
                        acce_ref.at[c, pl.ds(2 * r0, 2 * R16)])
        plsc.subcore_barrier()

        pltpu.sync_copy(zh_ref.at[pl.ds(2 * r0, 2 * R16)],
                        acchs.at[pl.ds(2 * r0, 2 * R16)])
        plsc.subcore_barrier()
        run_phase(True, compute_feat)
        plsc.subcore_barrier()
        pltpu.sync_copy(acchs.at[pl.ds(2 * r0, 2 * R16)],
                        acch_ref.at[c, pl.ds(2 * r0, 2 * R16)])

    return body


def _make_sc_layer(hc, mode_a):
    mesh = plsc.VectorSubcoreMesh(core_axis_name="c", subcore_axis_name="s")
    return pl.kernel(
        _sc_body_factory(hc, mode_a),
        out_type=(jax.ShapeDtypeStruct((NCORE, 2 * NP, 16), _f32),
                  jax.ShapeDtypeStruct((NCORE, 2 * NP, 16), _f32)),
        mesh=mesh,
        scratch_types=(
            [pltpu.VMEM((EB,), _i32)] * 12 +
            [pltpu.VMEM((EB, 16), _f32)] * 8 +
            [pltpu.VMEM_SHARED((2 * NP, 16), _f32)] +
            [pltpu.SemaphoreType.DMA] * 5
        ),
        compiler_params=pltpu.CompilerParams(needs_layout_passes=False,
                                             use_tc_tiling_on_sc=False),
        name=f"gat_edge_hc{hc}_{'A' if mode_a else 'B'}",
    )


def _full(shape):
    return pl.BlockSpec(shape, lambda i: tuple(0 for _ in shape))


def _rows(width):
    return pl.BlockSpec((RB, width), lambda i: (i, 0))


def _chunk_tabs(width):
    return pl.BlockSpec((2, RB, width), lambda i: (0, i, 0))


def _tables_4h(h, asr, adt):
    htaba = jnp.stack([h[:, 0:16], h[:, 32:48]])
    htabb = jnp.stack([h[:, 16:32], h[:, 48:64]])
    z6 = jnp.zeros((RB, 6), _f32)
    a0 = jnp.concatenate([asr[:, 0:2], z6, adt[:, 0:2], z6], axis=1)
    a1 = jnp.concatenate([asr[:, 2:4], z6, adt[:, 2:4], z6], axis=1)
    return htaba, htabb, jnp.stack([a0, a1])


def _k1_body(xf_ref, xb_ref, wf_ref, asf_ref, adf_ref,
             wb_ref, asb_ref, adb_ref,
             haf_ref, hbf_ref, af_ref, hab_ref, hbb_ref, ab_ref):
    for x_ref, w_ref, as_ref, ad_ref, ha_out, hb_out, a_out in (
            (xf_ref, wf_ref, asf_ref, adf_ref, haf_ref, hbf_ref, af_ref),
            (xb_ref, wb_ref, asb_ref, adb_ref, hab_ref, hbb_ref, ab_ref)):
        x = x_ref[...]
        h = jnp.dot(x, w_ref[...], preferred_element_type=_f32)
        asr = jnp.dot(h, as_ref[...], preferred_element_type=_f32)
        adt = jnp.dot(h, ad_ref[...], preferred_element_type=_f32)
        htaba, htabb, atab = _tables_4h(h, asr, adt)
        ha_out[...] = htaba
        hb_out[...] = htabb
        a_out[...] = atab


def _combine_4h(acch, acce, bias):
    cols = []
    for c in range(2):
        for hh in range(2):
            num = acch[c][:, hh * 16:(hh + 1) * 16]
            den = acce[c][:, hh:hh + 1] + 1e-16
            cols.append(num / den)
    v = jnp.concatenate(cols, axis=1) + bias
    return jnp.where(v > 0.0, v, jnp.exp(v) - 1.0)


def _k2_body(ahf_ref, aef_ref, ahb_ref, aeb_ref,
             bf_ref, wf_ref, asf_ref, adf_ref,
             bb_ref, wb_ref, asb_ref, adb_ref,
             haf_ref, hbf_ref, af_ref, hab_ref, hbb_ref, ab_ref):
    for ah_ref, ae_ref, b_ref, w_ref, as_ref, ad_ref, ha_out, hb_out, a_out in (
            (ahf_ref, aef_ref, bf_ref, wf_ref, asf_ref, adf_ref,
             haf_ref, hbf_ref, af_ref),
            (ahb_ref, aeb_ref, bb_ref, wb_ref, asb_ref, adb_ref,
             hab_ref, hbb_ref, ab_ref)):
        xin = _combine_4h(ah_ref[...], ae_ref[...], b_ref[...])
        h = jnp.dot(xin, w_ref[...], preferred_element_type=_f32)
        asr = jnp.dot(h, as_ref[...], preferred_element_type=_f32)
        adt = jnp.dot(h, ad_ref[...], preferred_element_type=_f32)
        htaba, htabb, atab = _tables_4h(h, asr, adt)
        ha_out[...] = htaba
        hb_out[...] = htabb
        a_out[...] = atab


def _k3_body(ahf_ref, aef_ref, ahb_ref, aeb_ref,
             bf_ref, wf_ref, asf_ref, adf_ref,
             bb_ref, wb_ref, asb_ref, adb_ref,
             haf_ref, hbf_ref, af_ref, hab_ref, hbb_ref, ab_ref):
    z7 = jnp.zeros((RB, 7), _f32)
    for ah_ref, ae_ref, b_ref, w_ref, as_ref, ad_ref, ha_out, hb_out, a_out in (
            (ahf_ref, aef_ref, bf_ref, wf_ref, asf_ref, adf_ref,
             haf_ref, hbf_ref, af_ref),
            (ahb_ref, aeb_ref, bb_ref, wb_ref, asb_ref, adb_ref,
             hab_ref, hbb_ref, ab_ref)):
        xin = _combine_4h(ah_ref[...], ae_ref[...], b_ref[...])
        h = jnp.dot(xin, w_ref[...], preferred_element_type=_f32)
        asr = jnp.dot(h, as_ref[...], preferred_element_type=_f32)
        adt = jnp.dot(h, ad_ref[...], preferred_element_type=_f32)
        ha_out[...] = h[:, 0:16]
        hb_out[...] = h[:, 16:32]
        a_out[...] = jnp.concatenate([asr, z7, adt, z7], axis=1)


def _kpool_body(ahf_ref, aef_ref, ahb_ref, aeb_ref,
                bf_ref, bb_ref, batch_ref,
                hnode_ref, gpool_ref, accp, accc):
    i = pl.program_id(0)
    parts = []
    for ah_ref, ae_ref, b_ref in ((ahf_ref, aef_ref, bf_ref),
                                  (ahb_ref, aeb_ref, bb_ref)):
        ah = ah_ref[...]
        ae = ae_ref[...]
        num = ah[0] + ah[1]
        den = ae[0][:, 0:1] + ae[1][:, 0:1] + 1e-16
        parts.append(num / den + b_ref[...])
    hn = jnp.concatenate(parts, axis=1)
    hnode_ref[...] = hn

    row = lax.broadcasted_iota(_i32, (RB, 1), 0) + i * RB
    valid = row < N
    bt = batch_ref[0, 0, :].reshape(RB, 1)
    gid = lax.broadcasted_iota(_i32, (RB, NGRAPH), 1)
    oh = jnp.where((bt == gid) & valid, 1.0, 0.0).astype(_f32)
    contrib = lax.dot_general(oh, hn, (((0,), (0,)), ((), ())),
                              preferred_element_type=_f32)
    cnt = jnp.sum(oh, axis=0).reshape(NGRAPH, 1)
    newp = jnp.where(i == 0, contrib, accp[...] + contrib)
    newc = jnp.where(i == 0, cnt, accc[...] + cnt)
    accp[...] = newp
    accc[...] = newc

    @pl.when(i == NBLK_TC - 1)
    def _():
        gpool_ref[...] = newp / jnp.clip(newc, 1.0)


def _run(xf, xb, src, dst, batchp, pp):
    zh = jnp.zeros((2 * NP, 16), _f32)

    tab4 = [jax.ShapeDtypeStruct((2, NP, 16), _f32)] * 3
    tab4_specs = [_chunk_tabs(16)] * 3

    k1 = pl.pallas_call(
        _k1_body,
        grid=(NBLK_TC,),
        in_specs=[_rows(8), _rows(8),
                  _full((8, 64)), _full((64, 4)), _full((64, 4)),
                  _full((8, 64)), _full((64, 4)), _full((64, 4))],
        out_specs=tab4_specs + tab4_specs,
        out_shape=tab4 + tab4,
    )
    haf1, hbf1, af1, hab1, hbb1, ab1 = k1(
        xf, xb, pp['f1W'], pp['f1As'], pp['f1Ad'],
        pp['b1W'], pp['b1As'], pp['b1Ad'])

    sc4 = _make_sc_layer(2, True)
    sc1 = _make_sc_layer(1, False)

    def run_sc(scfn, s_idx, d_idx, ha, hb, at):
        ah, ae = scfn(s_idx, d_idx, ha.reshape(-1, 16), hb.reshape(-1, 16),
                      at.reshape(-1, 16), zh)
        return ah.reshape(NCORE, NP, 32), ae.reshape(NCORE, NP, 32)

    ahf1, aef1 = run_sc(sc4, src, dst, haf1, hbf1, af1)
    ahb1, aeb1 = run_sc(sc4, dst, src, hab1, hbb1, ab1)

    k2 = pl.pallas_call(
        _k2_body,
        grid=(NBLK_TC,),
        in_specs=[_chunk_tabs(32), _chunk_tabs(32),
                  _chunk_tabs(32), _chunk_tabs(32),
                  _full((1, 64)), _full((64, 64)), _full((64, 4)), _full((64, 4)),
                  _full((1, 64)), _full((64, 64)), _full((64, 4)), _full((64, 4))],
        out_specs=tab4_specs + tab4_specs,
        out_shape=tab4 + tab4,
    )
    haf2, hbf2, af2, hab2, hbb2, ab2 = k2(
        ahf1, aef1, ahb1, aeb1,
        pp['f1b'], pp['f2W'], pp['f2As'], pp['f2Ad'],
        pp['b1b'], pp['b2W'], pp['b2As'], pp['b2Ad'])

    ahf2, aef2 = run_sc(sc4, src, dst, haf2, hbf2, af2)
    ahb2, aeb2 = run_sc(sc4, dst, src, hab2, hbb2, ab2)

    tab1 = [jax.ShapeDtypeStruct((NP, 16), _f32)] * 3
    tab1_specs = [_rows(16)] * 3
    k3 = pl.pallas_call(
        _k3_body,
        grid=(NBLK_TC,),
        in_specs=[_chunk_tabs(32), _chunk_tabs(32),
                  _chunk_tabs(32), _chunk_tabs(32),
                  _full((1, 64)), _full((64, 32)), _full((32, 1)), _full((32, 1)),
                  _full((1, 64)), _full((64, 32)), _full((32, 1)), _full((32, 1))],
        out_specs=tab1_specs + tab1_specs,
        out_shape=tab1 + tab1,
    )
    haf3, hbf3, af3, hab3, hbb3, ab3 = k3(
        ahf2, aef2, ahb2, aeb2,
        pp['f2b'], pp['f3W'], pp['f3As'], pp['f3Ad'],
        pp['b2b'], pp['b3W'], pp['b3As'], pp['b3Ad'])

    ahf3, aef3 = run_sc(sc1, src, dst, haf3, hbf3, af3)
    ahb3, aeb3 = run_sc(sc1, dst, src, hab3, hbb3, ab3)

    kpool = pl.pallas_call(
        _kpool_body,
        grid=(NBLK_TC,),
        in_specs=[_chunk_tabs(32), _chunk_tabs(32),
                  _chunk_tabs(32), _chunk_tabs(32),
                  _full((1, 32)), _full((1, 32)),
                  pl.BlockSpec((1, 1, RB), lambda i: (i, 0, 0))],
        out_specs=[_rows(64), pl.BlockSpec((NGRAPH, NGRAPH), lambda i: (0, 0))],
        out_shape=[jax.ShapeDtypeStruct((N, 64), _f32),
                   jax.ShapeDtypeStruct((NGRAPH, NGRAPH), _f32)],
        scratch_shapes=[pltpu.VMEM((NGRAPH, NGRAPH), _f32),
                        pltpu.VMEM((NGRAPH, 1), _f32)],
    )
    h_node, g_pool = kpool(ahf3, aef3, ahb3, aeb3,
                           pp['f3b'], pp['b3b'], batchp)
    return h_node, g_pool


def _attn_mat(a, heads, out_ch):
    if heads == 1:
        return a.reshape(out_ch, 1)
    eye = jnp.repeat(jnp.eye(heads, dtype=_f32), out_ch, axis=0)
    return a.reshape(heads * out_ch, 1) * eye


def kernel(x, params, edge_index, batch):
    ei = edge_index.astype(_i32)
    loops = jnp.arange(N, dtype=_i32)
    padv = jnp.full((EPADV,), N, _i32)
    src = jnp.concatenate([ei[0], loops, padv])
    dst = jnp.concatenate([ei[1], loops, padv])

    xf = jnp.pad(x[:, jnp.array([0, 1, 3])], ((0, NP - N), (0, 5)))
    xb = jnp.pad(x[:, jnp.array([0, 2, 4])], ((0, NP - N), (0, 5)))
    batchp = jnp.pad(batch.astype(_i32), (0, NP - N)).reshape(NBLK_TC, 1, RB)

    pp = {}
    for name, heads, out_ch in (('f1', 4, 16), ('f2', 4, 16), ('f3', 1, 32),
                                ('b1', 4, 16), ('b2', 4, 16), ('b3', 1, 32)):
        p = params[name]
        w = p['W']
        if w.shape[0] == 3:
            w = jnp.pad(w, ((0, 5), (0, 0)))
        pp[name + 'W'] = w
        pp[name + 'As'] = _attn_mat(p['a_s'], heads, out_ch)
        pp[name + 'Ad'] = _attn_mat(p['a_d'], heads, out_ch)
        pp[name + 'b'] = p['b'].reshape(1, -1)

    return _run(xf, xb, src, dst, batchp, pp)


# speedup vs baseline: 37.1171x; 1.0073x over previous
import jax
import jax.numpy as jnp
from jax import lax
from jax.experimental import pallas as pl
from jax.experimental.pallas import tpu as pltpu
from jax.experimental.pallas import tpu_sc as plsc

N = 50000
E = 800000
NGRAPH = 64

NP = 50048
RB = 128
NBLK_TC = NP // RB
EPAD = 802816
EB = 128
EPADV = EPAD + EB - E
NTILE = 16
NCORE = 2
R16 = NP // NTILE

_f32 = jnp.float32
_i32 = jnp.int32


def _sc_body_factory(hc, mode_a):
    def body(src_ref, dst_ref, htaba_ref, htabb_ref, atab_ref, zh_ref,
             acch_ref, acce_ref,
             sidx0, sidx1, didx0, didx1, sidxt0, sidxt1, didxt0, didxt1,
             d2a0, d2a1, d2b0, d2b1,
             hra0, hra1, hrb0, hrb1, ars0, ars1, ard0, ard1,
             acchs, semi, semg0, semg1, sems0, sems1):
        sidx = (sidx0, sidx1)
        didx = (didx0, didx1)
        sidxt = (sidxt0, sidxt1)
        didxt = (didxt0, didxt1)
        d2a = (d2a0, d2a1)
        d2b = (d2b0, d2b1)
        hra = (hra0, hra1)
        hrb = (hrb0, hrb1)
        ars = (ars0, ars1)
        ard = (ard0, ard1)
        semg = (semg0, semg1)
        sems = (sems0, sems1)

        c = lax.axis_index("c")
        s = lax.axis_index("s")
        r0 = s * R16
        iota16 = lax.iota(_i32, 16)
        zeros16 = jnp.zeros((16,), _f32)

        if mode_a:
            nblk = EPAD // NTILE // EB
            e_base = s * (EPAD // NTILE)
            tab_off = c * NP
        else:
            nblk = EPAD // (NTILE * NCORE) // EB
            wid = c * NTILE + s
            e_base = wid * (EPAD // (NTILE * NCORE))
            tab_off = None

        def idx_issue(ib, nb):
            base = e_base + ib * EB
            ci = pltpu.async_copy(src_ref.at[pl.ds(base, EB)], sidx[nb], semi)
            cj = pltpu.async_copy(dst_ref.at[pl.ds(base, EB)], didx[nb], semi)
            return ci, cj

        def idx_finish(ci, cj, nb):
            ci.wait()
            cj.wait()
            for g in range(EB // 16):
                sl = pl.ds(g * 16, 16)
                dv = didx[nb][sl]
                d2a[nb][sl] = dv * 2
                d2b[nb][sl] = dv * 2 + 1
                if mode_a:
                    sidxt[nb][sl] = sidx[nb][sl] + tab_off
                    didxt[nb][sl] = dv + tab_off

        def gsrc(b):
            return sidxt[b] if mode_a else sidx[b]

        def gdst(b):
            return didxt[b] if mode_a else didx[b]

        def gathers_issue(b, with_h):
            pltpu.async_copy(atab_ref.at[gsrc(b)], ars[b], semg[b])
            pltpu.async_copy(atab_ref.at[gdst(b)], ard[b], semg[b])
            if with_h:
                pltpu.async_copy(htaba_ref.at[gsrc(b)], hra[b], semg[b])
                pltpu.async_copy(htabb_ref.at[gsrc(b)], hrb[b], semg[b])

        def gathers_wait(b, with_h):
            pltpu.make_async_copy(atab_ref.at[gsrc(b)], ars[b], semg[b]).wait()
            pltpu.make_async_copy(atab_ref.at[gdst(b)], ard[b], semg[b]).wait()
            if with_h:
                pltpu.make_async_copy(htaba_ref.at[gsrc(b)], hra[b],
                                      semg[b]).wait()
                pltpu.make_async_copy(htabb_ref.at[gsrc(b)], hrb[b],
                                      semg[b]).wait()

        def scatter_issue(b, with_h):
            pltpu.async_copy(hra[b], acchs.at[d2a[b]], sems[b], add=True)
            if with_h:
                pltpu.async_copy(hrb[b], acchs.at[d2b[b]], sems[b], add=True)

        def scatter_wait(b, with_h):
            pltpu.make_async_copy(hra[b], acchs.at[d2a[b]], sems[b]).wait()
            if with_h:
                pltpu.make_async_copy(hrb[b], acchs.at[d2b[b]],
                                      sems[b]).wait()

        def edge_ex(g, b):
            eids = iota16 + g * 16
            exs = []
            for hh in range(hc):
                asv = plsc.load_gather(ars[b],
                                       [eids, jnp.full((16,), hh, _i32)])
                adv = plsc.load_gather(ard[b],
                                       [eids, jnp.full((16,), 8 + hh, _i32)])
                al = asv + adv
                al = jnp.where(al > 0.0, al, al * 0.2)
                exs.append(jnp.exp(al))
            return eids, exs

        def compute_denom(b):
            for g in range(EB // 16):
                eids, exs = edge_ex(g, b)
                for hh in range(hc):
                    plsc.store_scatter(hra[b],
                                       [eids, jnp.full((16,), hh, _i32)],
                                       exs[hh])

        def compute_feat(b):
            for g in range(EB // 16):
                eids, exs = edge_ex(g, b)
                for col in range(32):
                    buf = hra[b] if col < 16 else hrb[b]
                    colv = jnp.full((16,), col % 16, _i32)
                    hv = plsc.load_gather(buf, [eids, colv])
                    plsc.store_scatter(buf, [eids, colv],
                                       hv * exs[col * hc // 32])

        def run_phase(with_h, compute):
            ci, cj = idx_issue(0, 0)
            idx_finish(ci, cj, 0)
            gathers_issue(0, with_h)

            def step(ib, b, first):
                nb = 1 - b
                gathers_wait(b, with_h)
                if not first:
                    scatter_wait(nb, with_h)
                ci, cj = idx_issue(ib + 1, nb)
                compute(b)
                scatter_issue(b, with_h)
                idx_finish(ci, cj, nb)
                gathers_issue(nb, with_h)

            step(0, 0, True)
            step(1, 1, False)

            @pl.loop(2, nblk, step=2)
            def _pipe(it):
                step(it, 0, False)
                step(it + 1, 1, False)

            gathers_wait(0, with_h)
            scatter_wait(1, with_h)

        @pl.loop(0, EB)
        def _zrow(r):
            for b in range(2):
                hra[b][r, pl.ds(0, 16)] = zeros16

        pltpu.sync_copy(zh_ref.at[pl.ds(2 * r0, 2 * R16)],
                        acchs.at[pl.ds(2 * r0, 2 * R16)])
        plsc.subcore_barrier()
        run_phase(False, compute_denom)
        plsc.subcore_barrier()
        pltpu.sync_copy(acchs.at[pl.ds(2 * r0, 2 * R16)],
                        acce_ref.at[c, pl.ds(2 * r0, 2 * R16)])
        plsc.subcore_barrier()

        pltpu.sync_copy(zh_ref.at[pl.ds(2 * r0, 2 * R16)],
                        acchs.at[pl.ds(2 * r0, 2 * R16)])
        plsc.subcore_barrier()
        run_phase(True, compute_feat)
        plsc.subcore_barrier()
        pltpu.sync_copy(acchs.at[pl.ds(2 * r0, 2 * R16)],
                        acch_ref.at[c, pl.ds(2 * r0, 2 * R16)])

    return body


def _make_sc_layer(hc, mode_a):
    mesh = plsc.VectorSubcoreMesh(core_axis_name="c", subcore_axis_name="s")
    return pl.kernel(
        _sc_body_factory(hc, mode_a),
        out_type=(jax.ShapeDtypeStruct((NCORE, 2 * NP, 16), _f32),
                  jax.ShapeDtypeStruct((NCORE, 2 * NP, 16), _f32)),
        mesh=mesh,
        scratch_types=(
            [pltpu.VMEM((EB,), _i32)] * 12 +
            [pltpu.VMEM((EB, 16), _f32)] * 8 +
            [pltpu.VMEM_SHARED((2 * NP, 16), _f32)] +
            [pltpu.SemaphoreType.DMA] * 5
        ),
        compiler_params=pltpu.CompilerParams(needs_layout_passes=False,
                                             use_tc_tiling_on_sc=False),
        name=f"gat_edge_hc{hc}_{'A' if mode_a else 'B'}",
    )


def _full(shape):
    return pl.BlockSpec(shape, lambda i: tuple(0 for _ in shape))


def _rows(width):
    return pl.BlockSpec((RB, width), lambda i: (i, 0))


def _chunk_tabs(width):
    return pl.BlockSpec((2, RB, width), lambda i: (0, i, 0))


def _tables_4h(h, asr, adt):
    htaba = jnp.stack([h[:, 0:16], h[:, 32:48]])
    htabb = jnp.stack([h[:, 16:32], h[:, 48:64]])
    z6 = jnp.zeros((RB, 6), _f32)
    a0 = jnp.concatenate([asr[:, 0:2], z6, adt[:, 0:2], z6], axis=1)
    a1 = jnp.concatenate([asr[:, 2:4], z6, adt[:, 2:4], z6], axis=1)
    return htaba, htabb, jnp.stack([a0, a1])


def _k1_body(xf_ref, xb_ref, wf_ref, asf_ref, adf_ref,
             wb_ref, asb_ref, adb_ref,
             haf_ref, hbf_ref, af_ref, hab_ref, hbb_ref, ab_ref):
    for x_ref, w_ref, as_ref, ad_ref, ha_out, hb_out, a_out in (
            (xf_ref, wf_ref, asf_ref, adf_ref, haf_ref, hbf_ref, af_ref),
            (xb_ref, wb_ref, asb_ref, adb_ref, hab_ref, hbb_ref, ab_ref)):
        x = x_ref[...]
        h = jnp.dot(x, w_ref[...], preferred_element_type=_f32)
        asr = jnp.dot(h, as_ref[...], preferred_element_type=_f32)
        adt = jnp.dot(h, ad_ref[...], preferred_element_type=_f32)
        htaba, htabb, atab = _tables_4h(h, asr, adt)
        ha_out[...] = htaba
        hb_out[...] = htabb
        a_out[...] = atab


def _self_ex(atab, c, hh):
    al = atab[c][:, hh:hh + 1] + atab[c][:, 8 + hh:9 + hh]
    return jnp.exp(jnp.where(al > 0.0, al, al * 0.2))


def _combine_4h(acch, acce, hta, htb, atab, bias):
    cols = []
    for c in range(2):
        for hh in range(2):
            exs = _self_ex(atab, c, hh)
            hself = hta[c] if hh == 0 else htb[c]
            num = acch[c][:, hh * 16:(hh + 1) * 16] + exs * hself
            den = acce[c][:, hh:hh + 1] + exs + 1e-16
            cols.append(num / den)
    v = jnp.concatenate(cols, axis=1) + bias
    return jnp.where(v > 0.0, v, jnp.exp(v) - 1.0)


def _k2_body(ahf_ref, aef_ref, ahb_ref, aeb_ref,
             phaf_ref, phbf_ref, paf_ref, phab_ref, phbb_ref, pab_ref,
             bf_ref, wf_ref, asf_ref, adf_ref,
             bb_ref, wb_ref, asb_ref, adb_ref,
             haf_ref, hbf_ref, af_ref, hab_ref, hbb_ref, ab_ref):
    for (ah_ref, ae_ref, pha, phb, pa, b_ref, w_ref, as_ref, ad_ref,
         ha_out, hb_out, a_out) in (
            (ahf_ref, aef_ref, phaf_ref, phbf_ref, paf_ref,
             bf_ref, wf_ref, asf_ref, adf_ref,
             haf_ref, hbf_ref, af_ref),
            (ahb_ref, aeb_ref, phab_ref, phbb_ref, pab_ref,
             bb_ref, wb_ref, asb_ref, adb_ref,
             hab_ref, hbb_ref, ab_ref)):
        xin = _combine_4h(ah_ref[...], ae_ref[...], pha[...], phb[...],
                          pa[...], b_ref[...])
        h = jnp.dot(xin, w_ref[...], preferred_element_type=_f32)
        asr = jnp.dot(h, as_ref[...], preferred_element_type=_f32)
        adt = jnp.dot(h, ad_ref[...], preferred_element_type=_f32)
        htaba, htabb, atab = _tables_4h(h, asr, adt)
        ha_out[...] = htaba
        hb_out[...] = htabb
        a_out[...] = atab


def _k3_body(ahf_ref, aef_ref, ahb_ref, aeb_ref,
             phaf_ref, phbf_ref, paf_ref, phab_ref, phbb_ref, pab_ref,
             bf_ref, wf_ref, asf_ref, adf_ref,
             bb_ref, wb_ref, asb_ref, adb_ref,
             haf_ref, hbf_ref, af_ref, hab_ref, hbb_ref, ab_ref):
    z7 = jnp.zeros((RB, 7), _f32)
    for (ah_ref, ae_ref, pha, phb, pa, b_ref, w_ref, as_ref, ad_ref,
         ha_out, hb_out, a_out) in (
            (ahf_ref, aef_ref, phaf_ref, phbf_ref, paf_ref,
             bf_ref, wf_ref, asf_ref, adf_ref,
             haf_ref, hbf_ref, af_ref),
            (ahb_ref, aeb_ref, phab_ref, phbb_ref, pab_ref,
             bb_ref, wb_ref, asb_ref, adb_ref,
             hab_ref, hbb_ref, ab_ref)):
        xin = _combine_4h(ah_ref[...], ae_ref[...], pha[...], phb[...],
                          pa[...], b_ref[...])
        h = jnp.dot(xin, w_ref[...], preferred_element_type=_f32)
        asr = jnp.dot(h, as_ref[...], preferred_element_type=_f32)
        adt = jnp.dot(h, ad_ref[...], preferred_element_type=_f32)
        ha_out[...] = h[:, 0:16]
        hb_out[...] = h[:, 16:32]
        a_out[...] = jnp.concatenate([asr, z7, adt, z7], axis=1)


def _kpool_body(ahf_ref, aef_ref, ahb_ref, aeb_ref,
                phaf_ref, phbf_ref, paf_ref, phab_ref, phbb_ref, pab_ref,
                bf_ref, bb_ref, batch_ref,
                hnode_ref, gpool_ref, accp, accc):
    i = pl.program_id(0)
    parts = []
    for ah_ref, ae_ref, pha, phb, pa, b_ref in (
            (ahf_ref, aef_ref, phaf_ref, phbf_ref, paf_ref, bf_ref),
            (ahb_ref, aeb_ref, phab_ref, phbb_ref, pab_ref, bb_ref)):
        ah = ah_ref[...]
        ae = ae_ref[...]
        al = pa[...][:, 0:1] + pa[...][:, 8:9]
        exs = jnp.exp(jnp.where(al > 0.0, al, al * 0.2))
        hself = jnp.concatenate([pha[...], phb[...]], axis=1)
        num = ah[0] + ah[1] + exs * hself
        den = ae[0][:, 0:1] + ae[1][:, 0:1] + exs + 1e-16
        parts.append(num / den + b_ref[...])
    hn = jnp.concatenate(parts, axis=1)
    hnode_ref[...] = hn

    row = lax.broadcasted_iota(_i32, (RB, 1), 0) + i * RB
    valid = row < N
    bt = batch_ref[0, 0, :].reshape(RB, 1)
    gid = lax.broadcasted_iota(_i32, (RB, NGRAPH), 1)
    oh = jnp.where((bt == gid) & valid, 1.0, 0.0).astype(_f32)
    contrib = lax.dot_general(oh, hn, (((0,), (0,)), ((), ())),
                              preferred_element_type=_f32)
    cnt = jnp.sum(oh, axis=0).reshape(NGRAPH, 1)
    newp = jnp.where(i == 0, contrib, accp[...] + contrib)
    newc = jnp.where(i == 0, cnt, accc[...] + cnt)
    accp[...] = newp
    accc[...] = newc

    @pl.when(i == NBLK_TC - 1)
    def _():
        gpool_ref[...] = newp / jnp.clip(newc, 1.0)


def _run(xf, xb, src, dst, batchp, pp):
    zh = jnp.zeros((2 * NP, 16), _f32)

    tab4 = [jax.ShapeDtypeStruct((2, NP, 16), _f32)] * 3
    tab4_specs = [_chunk_tabs(16)] * 3

    k1 = pl.pallas_call(
        _k1_body,
        grid=(NBLK_TC,),
        in_specs=[_rows(8), _rows(8),
                  _full((8, 64)), _full((64, 4)), _full((64, 4)),
                  _full((8, 64)), _full((64, 4)), _full((64, 4))],
        out_specs=tab4_specs + tab4_specs,
        out_shape=tab4 + tab4,
    )
    haf1, hbf1, af1, hab1, hbb1, ab1 = k1(
        xf, xb, pp['f1W'], pp['f1As'], pp['f1Ad'],
        pp['b1W'], pp['b1As'], pp['b1Ad'])

    sc4 = _make_sc_layer(2, True)
    sc1 = _make_sc_layer(1, False)

    def run_sc(scfn, s_idx, d_idx, ha, hb, at):
        ah, ae = scfn(s_idx, d_idx, ha.reshape(-1, 16), hb.reshape(-1, 16),
                      at.reshape(-1, 16), zh)
        return ah.reshape(NCORE, NP, 32), ae.reshape(NCORE, NP, 32)

    ahf1, aef1 = run_sc(sc4, src, dst, haf1, hbf1, af1)
    ahb1, aeb1 = run_sc(sc4, dst, src, hab1, hbb1, ab1)

    k2 = pl.pallas_call(
        _k2_body,
        grid=(NBLK_TC,),
        in_specs=[_chunk_tabs(32), _chunk_tabs(32),
                  _chunk_tabs(32), _chunk_tabs(32)] +
                 [_chunk_tabs(16)] * 6 +
                 [_full((1, 64)), _full((64, 64)), _full((64, 4)), _full((64, 4)),
                  _full((1, 64)), _full((64, 64)), _full((64, 4)), _full((64, 4))],
        out_specs=tab4_specs + tab4_specs,
        out_shape=tab4 + tab4,
    )
    haf2, hbf2, af2, hab2, hbb2, ab2 = k2(
        ahf1, aef1, ahb1, aeb1,
        haf1, hbf1, af1, hab1, hbb1, ab1,
        pp['f1b'], pp['f2W'], pp['f2As'], pp['f2Ad'],
        pp['b1b'], pp['b2W'], pp['b2As'], pp['b2Ad'])

    ahf2, aef2 = run_sc(sc4, src, dst, haf2, hbf2, af2)
    ahb2, aeb2 = run_sc(sc4, dst, src, hab2, hbb2, ab2)

    tab1 = [jax.ShapeDtypeStruct((NP, 16), _f32)] * 3
    tab1_specs = [_rows(16)] * 3
    k3 = pl.pallas_call(
        _k3_body,
        grid=(NBLK_TC,),
        in_specs=[_chunk_tabs(32), _chunk_tabs(32),
                  _chunk_tabs(32), _chunk_tabs(32)] +
                 [_chunk_tabs(16)] * 6 +
                 [_full((1, 64)), _full((64, 32)), _full((32, 1)), _full((32, 1)),
                  _full((1, 64)), _full((64, 32)), _full((32, 1)), _full((32, 1))],
        out_specs=tab1_specs + tab1_specs,
        out_shape=tab1 + tab1,
    )
    haf3, hbf3, af3, hab3, hbb3, ab3 = k3(
        ahf2, aef2, ahb2, aeb2,
        haf2, hbf2, af2, hab2, hbb2, ab2,
        pp['f2b'], pp['f3W'], pp['f3As'], pp['f3Ad'],
        pp['b2b'], pp['b3W'], pp['b3As'], pp['b3Ad'])

    ahf3, aef3 = run_sc(sc1, src, dst, haf3, hbf3, af3)
    ahb3, aeb3 = run_sc(sc1, dst, src, hab3, hbb3, ab3)

    kpool = pl.pallas_call(
        _kpool_body,
        grid=(NBLK_TC,),
        in_specs=[_chunk_tabs(32), _chunk_tabs(32),
                  _chunk_tabs(32), _chunk_tabs(32)] +
                 [_rows(16)] * 6 +
                 [_full((1, 32)), _full((1, 32)),
                  pl.BlockSpec((1, 1, RB), lambda i: (i, 0, 0))],
        out_specs=[_rows(64), pl.BlockSpec((NGRAPH, NGRAPH), lambda i: (0, 0))],
        out_shape=[jax.ShapeDtypeStruct((N, 64), _f32),
                   jax.ShapeDtypeStruct((NGRAPH, NGRAPH), _f32)],
        scratch_shapes=[pltpu.VMEM((NGRAPH, NGRAPH), _f32),
                        pltpu.VMEM((NGRAPH, 1), _f32)],
    )
    h_node, g_pool = kpool(ahf3, aef3, ahb3, aeb3,
                           haf3, hbf3, af3, hab3, hbb3, ab3,
                           pp['f3b'], pp['b3b'], batchp)
    return h_node, g_pool


def _attn_mat(a, heads, out_ch):
    if heads == 1:
        return a.reshape(out_ch, 1)
    eye = jnp.repeat(jnp.eye(heads, dtype=_f32), out_ch, axis=0)
    return a.reshape(heads * out_ch, 1) * eye


def kernel(x, params, edge_index, batch):
    ei = edge_index.astype(_i32)
    padv = jnp.full((EPADV,), N, _i32)
    src = jnp.concatenate([ei[0], padv])
    dst = jnp.concatenate([ei[1], padv])

    xf = jnp.pad(x[:, jnp.array([0, 1, 3])], ((0, NP - N), (0, 5)))
    xb = jnp.pad(x[:, jnp.array([0, 2, 4])], ((0, NP - N), (0, 5)))
    batchp = jnp.pad(batch.astype(_i32), (0, NP - N)).reshape(NBLK_TC, 1, RB)

    pp = {}
    for name, heads, out_ch in (('f1', 4, 16), ('f2', 4, 16), ('f3', 1, 32),
                                ('b1', 4, 16), ('b2', 4, 16), ('b3', 1, 32)):
        p = params[name]
        w = p['W']
        if w.shape[0] == 3:
            w = jnp.pad(w, ((0, 5), (0, 0)))
        pp[name + 'W'] = w
        pp[name + 'As'] = _attn_mat(p['a_s'], heads, out_ch)
        pp[name + 'Ad'] = _attn_mat(p['a_d'], heads, out_ch)
        pp[name + 'b'] = p['b'].reshape(1, -1)

    return _run(xf, xb, src, dst, batchp, pp)
